# baseline (speedup 1.0000x reference)
import functools

import jax
import jax.numpy as jnp
from jax import lax
from jax.experimental import pallas as pl
from jax.experimental.pallas import tpu as pltpu
from jax.experimental.pallas import tpu_sc as plsc

NR = 10000
N = 10240
E = 320000
D_IN = 128
HEADS = 8
D_HEAD = 32
HID = 256
OUTC = 40
SLOPE = 0.2
NTRAIN = 5000

BN = 1024
GRID = N // BN
EPT = E // 32
EPT16 = E // 16

CH2 = 2000
CH3 = 80
CH2B = 2000
CH3B = 80

L2W = 128
NT = N // 16


def _entry_body(use_norm, hh, halfw, *refs):
    if use_norm:
        if hh == 8:
            (ra_ref, rb_ref, da_ref, db_ref, bprev_ref, w_ref, alx_ref,
             arx_ref, fa_ref, fb_ref, *score_refs) = refs
        else:
            (ra_ref, rb_ref, da_ref, db_ref, bprev_ref, w_ref, alx_ref,
             arx_ref, fa_ref, *score_refs) = refs
        den = jnp.concatenate([da_ref[...], db_ref[...]], axis=1)
        rden = jnp.where(den > 0.0, 1.0 / den, 0.0)
        raw = jnp.concatenate([ra_ref[...], rb_ref[...]], axis=1)
        rx = jnp.broadcast_to(rden[:, :, None], (BN, 8, raw.shape[1] // 8))
        h = jnp.maximum(raw * rx.reshape(BN, raw.shape[1]) + bprev_ref[...],
                        0.0)
    else:
        (x_ref, w_ref, alx_ref, arx_ref,
         fa_ref, fb_ref, *score_refs) = refs
        h = x_ref[...]
    i = pl.program_id(0)
    feat = jnp.dot(h, w_ref[...], preferred_element_type=jnp.float32)
    if hh == 8:
        fa_ref[...] = feat[:, :halfw]
        fb_ref[...] = feat[:, halfw:]
    else:
        fa_ref[...] = feat
    el = lax.dot_general(alx_ref[...], feat, (((1,), (1,)), ((), ())),
                         preferred_element_type=jnp.float32)
    er = lax.dot_general(arx_ref[...], feat, (((1,), (1,)), ((), ())),
                         preferred_element_type=jnp.float32)
    if hh == 8:
        ela_ref, elb_ref, era_ref, erb_ref, m_ref, acc_ref = score_refs
        ela_ref[:, pl.ds(i * BN, BN)] = el[:4]
        elb_ref[:, pl.ds(i * BN, BN)] = el[4:]
        era_ref[:, pl.ds(i * BN, BN)] = er[:4]
        erb_ref[:, pl.ds(i * BN, BN)] = er[4:]
    else:
        elt_ref, ert_ref, m_ref, acc_ref = score_refs
        elt_ref[:, pl.ds(i * BN, BN)] = el
        ert_ref[:, pl.ds(i * BN, BN)] = er
    bl = jnp.max(el)
    br = jnp.max(er)

    @pl.when(i == 0)
    def _():
        acc_ref[0] = bl
        acc_ref[1] = br

    @pl.when(i > 0)
    def _():
        acc_ref[0] = jnp.maximum(acc_ref[0], bl)
        acc_ref[1] = jnp.maximum(acc_ref[1], br)

    m = jnp.maximum(acc_ref[0] + acc_ref[1], 0.0)
    m_ref[...] = jnp.full((8, 128), m, jnp.float32)


def _tc_entry(use_norm, hh, fw, din, x_args, w, alx, arx):
    halfw = fw // 2
    in_specs = []
    if use_norm:
        in_specs += [
            pl.BlockSpec((BN, 128), lambda i: (i, 0)),
            pl.BlockSpec((BN, 128), lambda i: (i, 0)),
            pl.BlockSpec((BN, 4), lambda i: (i, 0)),
            pl.BlockSpec((BN, 4), lambda i: (i, 0)),
            pl.BlockSpec((1, 256), lambda i: (0, 0)),
        ]
    else:
        in_specs += [pl.BlockSpec((BN, din), lambda i: (i, 0))]
    in_specs += [
        pl.BlockSpec((din if not use_norm else 256, fw), lambda i: (0, 0)),
        pl.BlockSpec((hh, fw), lambda i: (0, 0)),
        pl.BlockSpec((hh, fw), lambda i: (0, 0)),
    ]
    nsc = 4 if hh == 8 else 2
    nf = 2 if hh == 8 else 1
    fwo = halfw if hh == 8 else fw
    hh2 = hh // 2 if hh == 8 else hh
    out_specs = (
        [pl.BlockSpec((BN, fwo), lambda i: (i, 0))] * nf
        + [pl.BlockSpec((hh2, N), lambda i: (0, 0))] * nsc
        + [pl.BlockSpec((8, 128), lambda i: (0, 0))]
    )
    out_shape = (
        [jax.ShapeDtypeStruct((N, fwo), jnp.float32)] * nf
        + [jax.ShapeDtypeStruct((hh2, N), jnp.float32)] * nsc
        + [jax.ShapeDtypeStruct((8, 128), jnp.float32)]
    )
    fn = pl.pallas_call(
        functools.partial(_entry_body, use_norm, hh, halfw),
        grid=(GRID,),
        in_specs=in_specs,
        out_specs=out_specs,
        out_shape=out_shape,
        scratch_shapes=[pltpu.SMEM((2,), jnp.float32)],
    )
    return fn(*x_args, w, alx, arx)


def _final_body(ra_ref, rb_ref, da_ref, db_ref, b2_ref, lab_ref, cnt_ref,
                logp_ref, loss_ref, acc_ref):
    i = pl.program_id(0)
    raw = ra_ref[...] + rb_ref[...]
    den = da_ref[...] + db_ref[...]
    rden = jnp.where(den > 0.0, 1.0 / den, 0.0)
    h = raw * rden + b2_ref[...]
    colmask = lax.broadcasted_iota(jnp.int32, (1, L2W), 1) < OUTC
    hm = jnp.where(colmask, h, -jnp.inf)
    mx = jnp.max(hm, axis=1, keepdims=True)
    ex = jnp.where(colmask, jnp.exp(h - mx), 0.0)
    lse = jnp.log(jnp.sum(ex, axis=1, keepdims=True)) + mx
    logp = h - lse
    logp_ref[...] = logp[:, :OUTC]
    lab = lab_ref[...]
    onehot = lax.broadcasted_iota(jnp.int32, (BN, L2W), 1) == lab
    pick = jnp.sum(jnp.where(onehot, logp, 0.0), axis=1)
    part = jnp.sum(pick * cnt_ref[...][:, 0])

    @pl.when(i == 0)
    def _():
        acc_ref[0] = part

    @pl.when(i > 0)
    def _():
        acc_ref[0] = acc_ref[0] + part

    loss_ref[...] = jnp.full((1, 1), -acc_ref[0] / float(NTRAIN),
                             jnp.float32)


def _tc_final(ra, rb, d0, d1, b2x, lab2d, cnt2d):
    fn = pl.pallas_call(
        _final_body,
        grid=(GRID,),
        in_specs=[
            pl.BlockSpec((BN, L2W), lambda i: (i, 0)),
            pl.BlockSpec((BN, L2W), lambda i: (i, 0)),
            pl.BlockSpec((BN, 1), lambda i: (i, 0)),
            pl.BlockSpec((BN, 1), lambda i: (i, 0)),
            pl.BlockSpec((1, L2W), lambda i: (0, 0)),
            pl.BlockSpec((BN, 1), lambda i: (i, 0)),
            pl.BlockSpec((BN, 1), lambda i: (i, 0)),
        ],
        out_specs=[
            pl.BlockSpec((BN, OUTC), lambda i: (i, 0)),
            pl.BlockSpec((1, 1), lambda i: (0, 0)),
        ],
        out_shape=[
            jax.ShapeDtypeStruct((N, OUTC), jnp.float32),
            jax.ShapeDtypeStruct((1, 1), jnp.float32),
        ],
        scratch_shapes=[pltpu.SMEM((1,), jnp.float32)],
    )
    return fn(ra, rb, d0, d1, b2x, lab2d, cnt2d)


def _mesh():
    return plsc.VectorSubcoreMesh(core_axis_name="c", subcore_axis_name="s",
                                  num_cores=2, num_subcores=16)


_CP = dict(compiler_params=pltpu.CompilerParams(needs_layout_passes=False))


def _sc_logits8(elaf, elbf, eraf, erbf, m, srcs, dsts, z4):
    @functools.partial(
        pl.kernel,
        out_type=(
            jax.ShapeDtypeStruct((4 * E,), jnp.float32),
            jax.ShapeDtypeStruct((4 * E,), jnp.float32),
            jax.ShapeDtypeStruct((4 * N,), jnp.float32),
            jax.ShapeDtypeStruct((4 * N,), jnp.float32),
        ),
        mesh=_mesh(), **_CP,
        scratch_types=[
            pltpu.VMEM((4 * N,), jnp.float32),
            pltpu.VMEM((4 * N,), jnp.float32),
            pltpu.VMEM((8, 128), jnp.float32),
            pltpu.VMEM((CH2,), jnp.int32),
            pltpu.VMEM((CH2,), jnp.int32),
            pltpu.VMEM((4 * CH2,), jnp.float32),
            pltpu.VMEM((4 * CH2,), jnp.int32),
            pltpu.VMEM_SHARED((4 * N,), jnp.float32),
        ],
    )
    def k(ela_ref, elb_ref, era_ref, erb_ref, m_ref, src_ref, dst_ref, z4_ref,
          ee0_ref, ee1_ref, den0_ref, den1_ref,
          elv, erv, mv, srcv, dstv, eec, didx, dacc):
        c = lax.axis_index("c")
        s = lax.axis_index("s")
        iota = lax.iota(jnp.int32, 16)
        iexp = iota >> 2
        ihead = iota & 3

        pltpu.sync_copy(z4_ref.at[pl.ds(s * 4 * NT, 4 * NT)],
                        elv.at[pl.ds(0, 4 * NT)])
        pltpu.sync_copy(elv.at[pl.ds(0, 4 * NT)],
                        dacc.at[pl.ds(s * 4 * NT, 4 * NT)])
        plsc.subcore_barrier()

        def work(el_in, er_in, ee_ref, den_ref):
            pltpu.sync_copy(el_in, elv)
            pltpu.sync_copy(er_in, erv)
            pltpu.sync_copy(m_ref, mv)
            mvec = mv[0, pl.ds(0, 16)]

            @pl.loop(0, EPT16 // CH2)
            def _(kk):
                off = s * EPT16 + kk * CH2
                pltpu.sync_copy(src_ref.at[pl.ds(off, CH2)], srcv)
                pltpu.sync_copy(dst_ref.at[pl.ds(off, CH2)], dstv)

                @pl.loop(0, CH2 // 16)
                def _(g):
                    for q in range(4):
                        eidx = g * 16 + q * 4 + iexp
                        s4 = plsc.load_gather(srcv, [eidx])
                        d4 = plsc.load_gather(dstv, [eidx])
                        a = plsc.load_gather(elv, [s4 + ihead * N])
                        b = plsc.load_gather(erv, [d4 + ihead * N])
                        x = a + b
                        e = jnp.maximum(x, SLOPE * x)
                        eec[pl.ds(g * 64 + q * 16, 16)] = jnp.exp(e - mvec)
                        didx[pl.ds(g * 64 + q * 16, 16)] = d4 * 4 + ihead

                pltpu.sync_copy(eec, ee_ref.at[pl.ds(off * 4, 4 * CH2)])
                pltpu.sync_copy(eec, dacc.at[didx], add=True)

            plsc.subcore_barrier()
            pltpu.sync_copy(dacc.at[pl.ds(s * 4 * NT, 4 * NT)],
                            elv.at[pl.ds(0, 4 * NT)])
            pltpu.sync_copy(elv.at[pl.ds(0, 4 * NT)],
                            den_ref.at[pl.ds(s * 4 * NT, 4 * NT)])

        @pl.when(c == 0)
        def _():
            work(ela_ref, era_ref, ee0_ref, den0_ref)

        @pl.when(c == 1)
        def _():
            work(elb_ref, erb_ref, ee1_ref, den1_ref)

    return k(elaf, elbf, eraf, erbf, m, srcs, dsts, z4)


def _sc_aggr8(srcs, dsts, ee0, ee1, fa, fb, z128):
    NCH = EPT16 // CH3

    @functools.partial(
        pl.kernel,
        out_type=(
            jax.ShapeDtypeStruct((N, 128), jnp.float32),
            jax.ShapeDtypeStruct((N, 128), jnp.float32),
        ),
        mesh=_mesh(), **_CP,
        scratch_types=[
            pltpu.VMEM((CH3,), jnp.int32),
            pltpu.VMEM((CH3,), jnp.int32),
            pltpu.VMEM((4 * CH3,), jnp.float32),
            pltpu.VMEM((CH3, 128), jnp.float32),
            pltpu.VMEM((CH3,), jnp.int32),
            pltpu.VMEM((CH3,), jnp.int32),
            pltpu.VMEM((4 * CH3,), jnp.float32),
            pltpu.VMEM((CH3, 128), jnp.float32),
            pltpu.VMEM((CH3, 128), jnp.float32),
            pltpu.VMEM_SHARED((N, 128), jnp.float32),
            pltpu.SemaphoreType.DMA,
            pltpu.SemaphoreType.DMA,
        ],
    )
    def k(src_ref, dst_ref, ee0_ref, ee1_ref, fa_ref, fb_ref, z_ref,
          ra_ref, rb_ref, srcv0, dstv0, eec0, rows0, srcv1, dstv1, eec1,
          rows1, scaled, acc, sem0, sem1):
        c = lax.axis_index("c")
        s = lax.axis_index("s")
        bufs = ((srcv0, dstv0, eec0, rows0, sem0),
                (srcv1, dstv1, eec1, rows1, sem1))

        pltpu.sync_copy(z_ref.at[pl.ds(0, 40)], scaled.at[pl.ds(0, 40)])

        @pl.loop(0, NT // 40)
        def _(k5):
            pltpu.sync_copy(scaled.at[pl.ds(0, 40)],
                            acc.at[pl.ds(s * NT + k5 * 40, 40)])
        plsc.subcore_barrier()

        def work(ee_ref, f_ref, r_ref):
            def load_issue(kchunk, par):
                sv, dv, ev, rv, sm = bufs[par]
                off = s * EPT16 + kchunk * CH3
                pltpu.sync_copy(src_ref.at[pl.ds(off, CH3)], sv)
                pltpu.sync_copy(dst_ref.at[pl.ds(off, CH3)], dv)
                pltpu.sync_copy(ee_ref.at[pl.ds(off * 4, 4 * CH3)], ev)

            def consume(par):
                sv, dv, ev, rv, sm = bufs[par]
                pltpu.async_copy(f_ref.at[sv], rv, sm).wait()

                @pl.loop(0, CH3 // 16)
                def _(g):
                    evx = [ev[pl.ds(g * 64 + q * 16, 16)] for q in range(4)]
                    for e16 in range(16):
                        q, rr = divmod(e16, 4)
                        eg = g * 16 + e16
                        a = [jnp.broadcast_to(evx[q][4 * rr + h], (16,))
                             for h in range(4)]
                        for j in range(8):
                            scaled[eg, pl.ds(j * 16, 16)] = (
                                rv[eg, pl.ds(j * 16, 16)] * a[j // 2])

                pltpu.sync_copy(scaled, acc.at[dv], add=True)

            load_issue(0, 0)

            @pl.loop(0, NCH // 2)
            def _(kk2):
                for par in range(2):
                    kchunk = kk2 * 2 + par

                    @pl.when(kchunk + 1 < NCH)
                    def _():
                        load_issue(kchunk + 1, 1 - par)

                    consume(par)

            plsc.subcore_barrier()

            @pl.loop(0, NT // 40)
            def _(k5):
                pltpu.sync_copy(acc.at[pl.ds(s * NT + k5 * 40, 40)],
                                rows0.at[pl.ds(0, 40)])
                pltpu.sync_copy(rows0.at[pl.ds(0, 40)],
                                r_ref.at[pl.ds(s * NT + k5 * 40, 40)])

        @pl.when(c == 0)
        def _():
            work(ee0_ref, fa_ref, ra_ref)

        @pl.when(c == 1)
        def _():
            work(ee1_ref, fb_ref, rb_ref)

    return k(srcs, dsts, ee0, ee1, fa, fb, z128)


def _sc_logits1(elt, ert, m, srcs, dsts, z1):
    @functools.partial(
        pl.kernel,
        out_type=(
            jax.ShapeDtypeStruct((E,), jnp.float32),
            jax.ShapeDtypeStruct((N,), jnp.float32),
            jax.ShapeDtypeStruct((N,), jnp.float32),
        ),
        mesh=_mesh(), **_CP,
        scratch_types=[
            pltpu.VMEM((N,), jnp.float32),
            pltpu.VMEM((N,), jnp.float32),
            pltpu.VMEM((8, 128), jnp.float32),
            pltpu.VMEM((CH2B,), jnp.int32),
            pltpu.VMEM((CH2B,), jnp.int32),
            pltpu.VMEM((CH2B,), jnp.float32),
            pltpu.VMEM_SHARED((N,), jnp.float32),
        ],
    )
    def k(elt_ref, ert_ref, m_ref, src_ref, dst_ref, z1_ref,
          ee_ref, den0_ref, den1_ref,
          elv, erv, mv, srcv, dstv, eec, dacc):
        c = lax.axis_index("c")
        s = lax.axis_index("s")

        pltpu.sync_copy(z1_ref.at[pl.ds(s * NT, NT)], elv.at[pl.ds(0, NT)])
        pltpu.sync_copy(elv.at[pl.ds(0, NT)], dacc.at[pl.ds(s * NT, NT)])
        plsc.subcore_barrier()

        pltpu.sync_copy(elt_ref, elv)
        pltpu.sync_copy(ert_ref, erv)
        pltpu.sync_copy(m_ref, mv)
        mvec = mv[0, pl.ds(0, 16)]

        @pl.loop(0, EPT // CH2B)
        def _(kk):
            off = c * (E // 2) + s * EPT + kk * CH2B
            pltpu.sync_copy(src_ref.at[pl.ds(off, CH2B)], srcv)
            pltpu.sync_copy(dst_ref.at[pl.ds(off, CH2B)], dstv)

            @pl.loop(0, CH2B // 16)
            def _(g):
                s16 = srcv[pl.ds(g * 16, 16)]
                d16 = dstv[pl.ds(g * 16, 16)]
                a = plsc.load_gather(elv, [s16])
                b = plsc.load_gather(erv, [d16])
                x = a + b
                e = jnp.maximum(x, SLOPE * x)
                eec[pl.ds(g * 16, 16)] = jnp.exp(e - mvec)

            pltpu.sync_copy(eec, ee_ref.at[pl.ds(off, CH2B)])
            pltpu.sync_copy(eec, dacc.at[dstv], add=True)

        plsc.subcore_barrier()

        for cc, den_ref in ((0, den0_ref), (1, den1_ref)):
            @pl.when(c == cc)
            def _(den_ref=den_ref):
                pltpu.sync_copy(dacc.at[pl.ds(s * NT, NT)],
                                elv.at[pl.ds(0, NT)])
                pltpu.sync_copy(elv.at[pl.ds(0, NT)],
                                den_ref.at[pl.ds(s * NT, NT)])

    return k(elt, ert, m, srcs, dsts, z1)


def _sc_aggr1(srcs, dsts, ee, f2p, z64):
    @functools.partial(
        pl.kernel,
        out_type=(
            jax.ShapeDtypeStruct((N, L2W), jnp.float32),
            jax.ShapeDtypeStruct((N, L2W), jnp.float32),
        ),
        mesh=_mesh(), **_CP,
        scratch_types=[
            pltpu.VMEM((CH3B,), jnp.int32),
            pltpu.VMEM((CH3B,), jnp.int32),
            pltpu.VMEM((CH3B,), jnp.float32),
            pltpu.VMEM((CH3B, L2W), jnp.float32),
            pltpu.VMEM((CH3B, L2W), jnp.float32),
            pltpu.VMEM_SHARED((N, L2W), jnp.float32),
            pltpu.SemaphoreType.DMA,
        ],
    )
    def k(src_ref, dst_ref, ee_ref, f_ref, z_ref,
          ra_ref, rb_ref, srcv, dstv, eec, rows, scaled, acc, sem):
        c = lax.axis_index("c")
        s = lax.axis_index("s")

        pltpu.sync_copy(z_ref.at[pl.ds(0, 40)], scaled.at[pl.ds(0, 40)])

        @pl.loop(0, NT // 40)
        def _(k5):
            pltpu.sync_copy(scaled.at[pl.ds(0, 40)],
                            acc.at[pl.ds(s * NT + k5 * 40, 40)])
        plsc.subcore_barrier()

        def work(cc, r_ref):
            @pl.loop(0, EPT // CH3B)
            def _(kk):
                off = cc * (E // 2) + s * EPT + kk * CH3B
                pltpu.sync_copy(src_ref.at[pl.ds(off, CH3B)], srcv)
                pltpu.sync_copy(dst_ref.at[pl.ds(off, CH3B)], dstv)
                pltpu.sync_copy(ee_ref.at[pl.ds(off, CH3B)], eec)
                pltpu.async_copy(f_ref.at[srcv], rows, sem).wait()

                @pl.loop(0, CH3B // 16)
                def _(g):
                    ev = eec[pl.ds(g * 16, 16)]
                    for e16 in range(16):
                        eg = g * 16 + e16
                        a = jnp.broadcast_to(ev[e16], (16,))
                        for j in range(L2W // 16):
                            scaled[eg, pl.ds(j * 16, 16)] = (
                                rows[eg, pl.ds(j * 16, 16)] * a)

                pltpu.sync_copy(scaled, acc.at[dstv], add=True)

            plsc.subcore_barrier()

            @pl.loop(0, NT // 40)
            def _(k5):
                pltpu.sync_copy(acc.at[pl.ds(s * NT + k5 * 40, 40)],
                                rows.at[pl.ds(0, 40)])
                pltpu.sync_copy(rows.at[pl.ds(0, 40)],
                                r_ref.at[pl.ds(s * NT + k5 * 40, 40)])

        @pl.when(c == 0)
        def _():
            work(0, ra_ref)

        @pl.when(c == 1)
        def _():
            work(1, rb_ref)

    return k(srcs, dsts, ee, f2p, z64)


def _sc_count(train_nodes, z1):
    @functools.partial(
        pl.kernel,
        out_type=jax.ShapeDtypeStruct((N,), jnp.float32),
        mesh=_mesh(), **_CP,
        scratch_types=[
            pltpu.VMEM((NTRAIN,), jnp.int32),
            pltpu.VMEM((5008,), jnp.float32),
            pltpu.VMEM((N,), jnp.float32),
            pltpu.VMEM_SHARED((N,), jnp.float32),
        ],
    )
    def k(tn_ref, z1_ref, cnt_ref, tnv, ones, zstage, cacc):
        c = lax.axis_index("c")
        s = lax.axis_index("s")

        @pl.when((c == 0) & (s == 0))
        def _():
            pltpu.sync_copy(z1_ref, zstage)
            pltpu.sync_copy(zstage, cacc)
            pltpu.sync_copy(tn_ref, tnv)

            @pl.loop(0, 313)
            def _(i):
                ones[pl.ds(i * 16, 16)] = jnp.ones((16,), jnp.float32)

            pltpu.sync_copy(ones.at[pl.ds(0, NTRAIN)], cacc.at[tnv],
                            add=True)
            pltpu.sync_copy(cacc, zstage)
            pltpu.sync_copy(zstage, cnt_ref)

    return k(train_nodes, z1)


def _blockdiag(a, hh, dh, fw):
    eye = jnp.eye(hh, dtype=a.dtype)
    out = (a[:, None, :] * eye[:, :, None]).reshape(hh, hh * dh)
    if out.shape[1] < fw:
        out = jnp.pad(out, ((0, 0), (0, fw - out.shape[1])))
    return out


def kernel(feats, edge_index, label, train_nodes, W0, al0, ar0, b0,
           W1, al1, ar1, b1, W2, al2, ar2, b2):
    edge = edge_index.astype(jnp.int32)
    srcs = edge[0]
    dsts = edge[1]

    alx0 = _blockdiag(al0, HEADS, D_HEAD, HID)
    arx0 = _blockdiag(ar0, HEADS, D_HEAD, HID)
    alx1 = _blockdiag(al1, HEADS, D_HEAD, HID)
    arx1 = _blockdiag(ar1, HEADS, D_HEAD, HID)
    W2x = jnp.pad(W2, ((0, 0), (0, L2W - OUTC)))
    alx2 = jnp.pad(al2, ((0, 0), (0, L2W - OUTC)))
    arx2 = jnp.pad(ar2, ((0, 0), (0, L2W - OUTC)))
    b2x = jnp.pad(b2, (0, L2W - OUTC)).reshape(1, L2W)
    b0r = b0.reshape(1, HID)
    b1r = b1.reshape(1, HID)

    featsp = jnp.pad(feats, ((0, N - NR), (0, 0)))
    labelp = jnp.pad(label.astype(jnp.int32), (0, N - NR))

    z4 = jnp.zeros((4 * N,), jnp.float32)
    z128 = jnp.zeros((N, 128), jnp.float32)
    z1 = jnp.zeros((N,), jnp.float32)
    z64 = jnp.zeros((N, L2W), jnp.float32)

    fa, fb, ela, elb, era, erb, m = _tc_entry(False, HEADS, HID, D_IN,
                                              (featsp,), W0, alx0, arx0)
    ee0, ee1, d0, d1 = _sc_logits8(ela.reshape(4 * N), elb.reshape(4 * N),
                                   era.reshape(4 * N), erb.reshape(4 * N),
                                   m, srcs, dsts, z4)
    ra, rb = _sc_aggr8(srcs, dsts, ee0, ee1, fa, fb, z128)

    fa, fb, ela, elb, era, erb, m = _tc_entry(True, HEADS, HID, D_IN,
                                              (ra, rb, d0.reshape(N, 4),
                                               d1.reshape(N, 4), b0r),
                                              W1, alx1, arx1)
    ee0, ee1, d0, d1 = _sc_logits8(ela.reshape(4 * N), elb.reshape(4 * N),
                                   era.reshape(4 * N), erb.reshape(4 * N),
                                   m, srcs, dsts, z4)
    ra, rb = _sc_aggr8(srcs, dsts, ee0, ee1, fa, fb, z128)

    f2, elt, ert, m = _tc_entry(True, 1, L2W, D_IN,
                                (ra, rb, d0.reshape(N, 4),
                                 d1.reshape(N, 4), b1r),
                                W2x, alx2, arx2)
    ee, dn0, dn1 = _sc_logits1(elt.reshape(N), ert.reshape(N), m,
                               srcs, dsts, z1)
    ra, rb = _sc_aggr1(srcs, dsts, ee, f2, z64)

    cnt = _sc_count(train_nodes.astype(jnp.int32), z1)

    logp, loss = _tc_final(ra, rb, dn0.reshape(N, 1), dn1.reshape(N, 1),
                           b2x, labelp.reshape(N, 1), cnt.reshape(N, 1))
    return logp[:NR], loss[0, 0]


# speedup vs baseline: 1.2424x; 1.2424x over previous
import functools

import jax
import jax.numpy as jnp
from jax import lax
from jax.experimental import pallas as pl
from jax.experimental.pallas import tpu as pltpu
from jax.experimental.pallas import tpu_sc as plsc

NR = 10000
N = 10240
E = 320000
D_IN = 128
HEADS = 8
D_HEAD = 32
HID = 256
OUTC = 40
SLOPE = 0.2
NTRAIN = 5000

BN = 1024
GRID = N // BN
EPT = E // 32
EPT16 = E // 16

CH2 = 2000
CH3 = 80
CH2B = 2000
CH3B = 80

L2W = 128
NT = N // 16


def _entry_body(use_norm, hh, halfw, *refs):
    if use_norm:
        if hh == 8:
            (ra_ref, rb_ref, da_ref, db_ref, bprev_ref, w_ref, alx_ref,
             arx_ref, fa_ref, fb_ref, *score_refs) = refs
        else:
            (ra_ref, rb_ref, da_ref, db_ref, bprev_ref, w_ref, alx_ref,
             arx_ref, fa_ref, *score_refs) = refs
        den = jnp.concatenate([da_ref[...], db_ref[...]], axis=1)
        rden = jnp.where(den > 0.0, 1.0 / den, 0.0)
        raw = jnp.concatenate([ra_ref[...], rb_ref[...]], axis=1)
        rx = jnp.broadcast_to(rden[:, :, None], (BN, 8, raw.shape[1] // 8))
        h = jnp.maximum(raw * rx.reshape(BN, raw.shape[1]) + bprev_ref[...],
                        0.0)
    else:
        (x_ref, w_ref, alx_ref, arx_ref,
         fa_ref, fb_ref, *score_refs) = refs
        h = x_ref[...]
    i = pl.program_id(0)
    feat = jnp.dot(h, w_ref[...], preferred_element_type=jnp.float32)
    if hh == 8:
        fa_ref[...] = feat[:, :halfw]
        fb_ref[...] = feat[:, halfw:]
    else:
        fa_ref[...] = feat
    el = lax.dot_general(alx_ref[...], feat, (((1,), (1,)), ((), ())),
                         preferred_element_type=jnp.float32)
    er = lax.dot_general(arx_ref[...], feat, (((1,), (1,)), ((), ())),
                         preferred_element_type=jnp.float32)
    if hh == 8:
        ela_ref, elb_ref, era_ref, erb_ref, m_ref, acc_ref = score_refs
        ela_ref[:, pl.ds(i * BN, BN)] = el[:4]
        elb_ref[:, pl.ds(i * BN, BN)] = el[4:]
        era_ref[:, pl.ds(i * BN, BN)] = er[:4]
        erb_ref[:, pl.ds(i * BN, BN)] = er[4:]
    else:
        elt_ref, ert_ref, m_ref, acc_ref = score_refs
        elt_ref[:, pl.ds(i * BN, BN)] = el
        ert_ref[:, pl.ds(i * BN, BN)] = er
    bl = jnp.max(el)
    br = jnp.max(er)

    @pl.when(i == 0)
    def _():
        acc_ref[0] = bl
        acc_ref[1] = br

    @pl.when(i > 0)
    def _():
        acc_ref[0] = jnp.maximum(acc_ref[0], bl)
        acc_ref[1] = jnp.maximum(acc_ref[1], br)

    m = jnp.maximum(acc_ref[0] + acc_ref[1], 0.0)
    m_ref[...] = jnp.full((8, 128), m, jnp.float32)


def _tc_entry(use_norm, hh, fw, din, x_args, w, alx, arx):
    halfw = fw // 2
    in_specs = []
    if use_norm:
        in_specs += [
            pl.BlockSpec((BN, 128), lambda i: (i, 0)),
            pl.BlockSpec((BN, 128), lambda i: (i, 0)),
            pl.BlockSpec((BN, 4), lambda i: (i, 0)),
            pl.BlockSpec((BN, 4), lambda i: (i, 0)),
            pl.BlockSpec((1, 256), lambda i: (0, 0)),
        ]
    else:
        in_specs += [pl.BlockSpec((BN, din), lambda i: (i, 0))]
    in_specs += [
        pl.BlockSpec((din if not use_norm else 256, fw), lambda i: (0, 0)),
        pl.BlockSpec((hh, fw), lambda i: (0, 0)),
        pl.BlockSpec((hh, fw), lambda i: (0, 0)),
    ]
    nsc = 4 if hh == 8 else 2
    nf = 2 if hh == 8 else 1
    fwo = halfw if hh == 8 else fw
    hh2 = hh // 2 if hh == 8 else hh
    out_specs = (
        [pl.BlockSpec((BN, fwo), lambda i: (i, 0))] * nf
        + [pl.BlockSpec((hh2, N), lambda i: (0, 0))] * nsc
        + [pl.BlockSpec((8, 128), lambda i: (0, 0))]
    )
    out_shape = (
        [jax.ShapeDtypeStruct((N, fwo), jnp.float32)] * nf
        + [jax.ShapeDtypeStruct((hh2, N), jnp.float32)] * nsc
        + [jax.ShapeDtypeStruct((8, 128), jnp.float32)]
    )
    fn = pl.pallas_call(
        functools.partial(_entry_body, use_norm, hh, halfw),
        grid=(GRID,),
        in_specs=in_specs,
        out_specs=out_specs,
        out_shape=out_shape,
        scratch_shapes=[pltpu.SMEM((2,), jnp.float32)],
    )
    return fn(*x_args, w, alx, arx)


def _final_body(ra_ref, rb_ref, da_ref, db_ref, b2_ref, lab_ref, cnt_ref,
                logp_ref, loss_ref, acc_ref):
    i = pl.program_id(0)
    raw = ra_ref[...] + rb_ref[...]
    den = da_ref[...] + db_ref[...]
    rden = jnp.where(den > 0.0, 1.0 / den, 0.0)
    h = raw * rden + b2_ref[...]
    colmask = lax.broadcasted_iota(jnp.int32, (1, L2W), 1) < OUTC
    hm = jnp.where(colmask, h, -jnp.inf)
    mx = jnp.max(hm, axis=1, keepdims=True)
    ex = jnp.where(colmask, jnp.exp(h - mx), 0.0)
    lse = jnp.log(jnp.sum(ex, axis=1, keepdims=True)) + mx
    logp = h - lse
    logp_ref[...] = logp[:, :OUTC]
    lab = lab_ref[...]
    onehot = lax.broadcasted_iota(jnp.int32, (BN, L2W), 1) == lab
    pick = jnp.sum(jnp.where(onehot, logp, 0.0), axis=1)
    part = jnp.sum(pick * cnt_ref[...][:, 0])

    @pl.when(i == 0)
    def _():
        acc_ref[0] = part

    @pl.when(i > 0)
    def _():
        acc_ref[0] = acc_ref[0] + part

    loss_ref[...] = jnp.full((1, 1), -acc_ref[0] / float(NTRAIN),
                             jnp.float32)


def _tc_final(ra, rb, d0, d1, b2x, lab2d, cnt2d):
    fn = pl.pallas_call(
        _final_body,
        grid=(GRID,),
        in_specs=[
            pl.BlockSpec((BN, L2W), lambda i: (i, 0)),
            pl.BlockSpec((BN, L2W), lambda i: (i, 0)),
            pl.BlockSpec((BN, 1), lambda i: (i, 0)),
            pl.BlockSpec((BN, 1), lambda i: (i, 0)),
            pl.BlockSpec((1, L2W), lambda i: (0, 0)),
            pl.BlockSpec((BN, 1), lambda i: (i, 0)),
            pl.BlockSpec((BN, 1), lambda i: (i, 0)),
        ],
        out_specs=[
            pl.BlockSpec((BN, OUTC), lambda i: (i, 0)),
            pl.BlockSpec((1, 1), lambda i: (0, 0)),
        ],
        out_shape=[
            jax.ShapeDtypeStruct((N, OUTC), jnp.float32),
            jax.ShapeDtypeStruct((1, 1), jnp.float32),
        ],
        scratch_shapes=[pltpu.SMEM((1,), jnp.float32)],
    )
    return fn(ra, rb, d0, d1, b2x, lab2d, cnt2d)


def _mesh():
    return plsc.VectorSubcoreMesh(core_axis_name="c", subcore_axis_name="s",
                                  num_cores=2, num_subcores=16)


_CP = dict(compiler_params=pltpu.CompilerParams(needs_layout_passes=False))


def _sc_logits8(elaf, elbf, eraf, erbf, m, srcs, dsts, z4):
    @functools.partial(
        pl.kernel,
        out_type=(
            jax.ShapeDtypeStruct((4 * E,), jnp.float32),
            jax.ShapeDtypeStruct((4 * E,), jnp.float32),
            jax.ShapeDtypeStruct((4 * N,), jnp.float32),
            jax.ShapeDtypeStruct((4 * N,), jnp.float32),
        ),
        mesh=_mesh(), **_CP,
        scratch_types=[
            pltpu.VMEM((4 * N,), jnp.float32),
            pltpu.VMEM((4 * N,), jnp.float32),
            pltpu.VMEM((8, 128), jnp.float32),
            pltpu.VMEM((CH2,), jnp.int32),
            pltpu.VMEM((CH2,), jnp.int32),
            pltpu.VMEM((4 * CH2,), jnp.float32),
            pltpu.VMEM((4 * CH2,), jnp.int32),
            pltpu.VMEM_SHARED((4 * N,), jnp.float32),
        ],
    )
    def k(ela_ref, elb_ref, era_ref, erb_ref, m_ref, src_ref, dst_ref, z4_ref,
          ee0_ref, ee1_ref, den0_ref, den1_ref,
          elv, erv, mv, srcv, dstv, eec, didx, dacc):
        c = lax.axis_index("c")
        s = lax.axis_index("s")
        iota = lax.iota(jnp.int32, 16)
        iexp = iota >> 2
        ihead = iota & 3

        pltpu.sync_copy(z4_ref.at[pl.ds(s * 4 * NT, 4 * NT)],
                        elv.at[pl.ds(0, 4 * NT)])
        pltpu.sync_copy(elv.at[pl.ds(0, 4 * NT)],
                        dacc.at[pl.ds(s * 4 * NT, 4 * NT)])
        plsc.subcore_barrier()

        def work(el_in, er_in, ee_ref, den_ref):
            pltpu.sync_copy(el_in, elv)
            pltpu.sync_copy(er_in, erv)
            pltpu.sync_copy(m_ref, mv)
            mvec = mv[0, pl.ds(0, 16)]

            @pl.loop(0, EPT16 // CH2)
            def _(kk):
                off = s * EPT16 + kk * CH2
                pltpu.sync_copy(src_ref.at[pl.ds(off, CH2)], srcv)
                pltpu.sync_copy(dst_ref.at[pl.ds(off, CH2)], dstv)

                @pl.loop(0, CH2 // 16)
                def _(g):
                    for q in range(4):
                        eidx = g * 16 + q * 4 + iexp
                        s4 = plsc.load_gather(srcv, [eidx])
                        d4 = plsc.load_gather(dstv, [eidx])
                        a = plsc.load_gather(elv, [s4 + ihead * N])
                        b = plsc.load_gather(erv, [d4 + ihead * N])
                        x = a + b
                        e = jnp.maximum(x, SLOPE * x)
                        eec[pl.ds(g * 64 + q * 16, 16)] = jnp.exp(e - mvec)
                        didx[pl.ds(g * 64 + q * 16, 16)] = d4 * 4 + ihead

                pltpu.sync_copy(eec, ee_ref.at[pl.ds(off * 4, 4 * CH2)])
                pltpu.sync_copy(eec, dacc.at[didx], add=True)

            plsc.subcore_barrier()
            pltpu.sync_copy(dacc.at[pl.ds(s * 4 * NT, 4 * NT)],
                            elv.at[pl.ds(0, 4 * NT)])
            pltpu.sync_copy(elv.at[pl.ds(0, 4 * NT)],
                            den_ref.at[pl.ds(s * 4 * NT, 4 * NT)])

        @pl.when(c == 0)
        def _():
            work(ela_ref, era_ref, ee0_ref, den0_ref)

        @pl.when(c == 1)
        def _():
            work(elb_ref, erb_ref, ee1_ref, den1_ref)

    return k(elaf, elbf, eraf, erbf, m, srcs, dsts, z4)


def _sc_aggr8(srcs, dsts, ee0, ee1, fa, fb, z128):
    NCH = EPT16 // CH3

    @functools.partial(
        pl.kernel,
        out_type=(
            jax.ShapeDtypeStruct((N, 128), jnp.float32),
            jax.ShapeDtypeStruct((N, 128), jnp.float32),
        ),
        mesh=_mesh(), **_CP,
        scratch_types=[
            pltpu.VMEM((CH3,), jnp.int32),
            pltpu.VMEM((CH3,), jnp.int32),
            pltpu.VMEM((4 * CH3,), jnp.float32),
            pltpu.VMEM((CH3, 128), jnp.float32),
            pltpu.VMEM((CH3,), jnp.int32),
            pltpu.VMEM((CH3,), jnp.int32),
            pltpu.VMEM((4 * CH3,), jnp.float32),
            pltpu.VMEM((CH3, 128), jnp.float32),
            pltpu.VMEM((CH3, 128), jnp.float32),
            pltpu.VMEM_SHARED((N, 128), jnp.float32),
            pltpu.SemaphoreType.DMA,
            pltpu.SemaphoreType.DMA,
        ],
    )
    def k(src_ref, dst_ref, ee0_ref, ee1_ref, fa_ref, fb_ref, z_ref,
          ra_ref, rb_ref, srcv0, dstv0, eec0, rows0, srcv1, dstv1, eec1,
          rows1, scaled, acc, sem0, sem1):
        c = lax.axis_index("c")
        s = lax.axis_index("s")
        bufs = ((srcv0, dstv0, eec0, rows0, sem0),
                (srcv1, dstv1, eec1, rows1, sem1))

        pltpu.sync_copy(z_ref.at[pl.ds(0, 40)], scaled.at[pl.ds(0, 40)])

        @pl.loop(0, NT // 40)
        def _(k5):
            pltpu.sync_copy(scaled.at[pl.ds(0, 40)],
                            acc.at[pl.ds(s * NT + k5 * 40, 40)])
        plsc.subcore_barrier()

        def work(ee_ref, f_ref, r_ref):
            def load_issue(kchunk, par):
                sv, dv, ev, rv, sm = bufs[par]
                off = s * EPT16 + kchunk * CH3
                pltpu.sync_copy(src_ref.at[pl.ds(off, CH3)], sv)
                pltpu.sync_copy(dst_ref.at[pl.ds(off, CH3)], dv)
                pltpu.sync_copy(ee_ref.at[pl.ds(off * 4, 4 * CH3)], ev)
                pltpu.async_copy(f_ref.at[sv], rv, sm)

            def consume(par):
                sv, dv, ev, rv, sm = bufs[par]
                pltpu.make_async_copy(f_ref.at[sv], rv, sm).wait()

                @pl.loop(0, CH3 // 16)
                def _(g):
                    evx = [ev[pl.ds(g * 64 + q * 16, 16)] for q in range(4)]
                    for e16 in range(16):
                        q, rr = divmod(e16, 4)
                        eg = g * 16 + e16
                        a = [jnp.broadcast_to(evx[q][4 * rr + h], (16,))
                             for h in range(4)]
                        for j in range(8):
                            scaled[eg, pl.ds(j * 16, 16)] = (
                                rv[eg, pl.ds(j * 16, 16)] * a[j // 2])

                pltpu.sync_copy(scaled, acc.at[dv], add=True)

            load_issue(0, 0)

            @pl.loop(0, NCH // 2)
            def _(kk2):
                for par in range(2):
                    kchunk = kk2 * 2 + par

                    @pl.when(kchunk + 1 < NCH)
                    def _():
                        load_issue(kchunk + 1, 1 - par)

                    consume(par)

            plsc.subcore_barrier()

            @pl.loop(0, NT // 40)
            def _(k5):
                pltpu.sync_copy(acc.at[pl.ds(s * NT + k5 * 40, 40)],
                                rows0.at[pl.ds(0, 40)])
                pltpu.sync_copy(rows0.at[pl.ds(0, 40)],
                                r_ref.at[pl.ds(s * NT + k5 * 40, 40)])

        @pl.when(c == 0)
        def _():
            work(ee0_ref, fa_ref, ra_ref)

        @pl.when(c == 1)
        def _():
            work(ee1_ref, fb_ref, rb_ref)

    return k(srcs, dsts, ee0, ee1, fa, fb, z128)


def _sc_logits1(elt, ert, m, srcs, dsts, z1):
    @functools.partial(
        pl.kernel,
        out_type=(
            jax.ShapeDtypeStruct((E,), jnp.float32),
            jax.ShapeDtypeStruct((N,), jnp.float32),
            jax.ShapeDtypeStruct((N,), jnp.float32),
        ),
        mesh=_mesh(), **_CP,
        scratch_types=[
            pltpu.VMEM((N,), jnp.float32),
            pltpu.VMEM((N,), jnp.float32),
            pltpu.VMEM((8, 128), jnp.float32),
            pltpu.VMEM((CH2B,), jnp.int32),
            pltpu.VMEM((CH2B,), jnp.int32),
            pltpu.VMEM((CH2B,), jnp.float32),
            pltpu.VMEM_SHARED((N,), jnp.float32),
        ],
    )
    def k(elt_ref, ert_ref, m_ref, src_ref, dst_ref, z1_ref,
          ee_ref, den0_ref, den1_ref,
          elv, erv, mv, srcv, dstv, eec, dacc):
        c = lax.axis_index("c")
        s = lax.axis_index("s")

        pltpu.sync_copy(z1_ref.at[pl.ds(s * NT, NT)], elv.at[pl.ds(0, NT)])
        pltpu.sync_copy(elv.at[pl.ds(0, NT)], dacc.at[pl.ds(s * NT, NT)])
        plsc.subcore_barrier()

        pltpu.sync_copy(elt_ref, elv)
        pltpu.sync_copy(ert_ref, erv)
        pltpu.sync_copy(m_ref, mv)
        mvec = mv[0, pl.ds(0, 16)]

        @pl.loop(0, EPT // CH2B)
        def _(kk):
            off = c * (E // 2) + s * EPT + kk * CH2B
            pltpu.sync_copy(src_ref.at[pl.ds(off, CH2B)], srcv)
            pltpu.sync_copy(dst_ref.at[pl.ds(off, CH2B)], dstv)

            @pl.loop(0, CH2B // 16)
            def _(g):
                s16 = srcv[pl.ds(g * 16, 16)]
                d16 = dstv[pl.ds(g * 16, 16)]
                a = plsc.load_gather(elv, [s16])
                b = plsc.load_gather(erv, [d16])
                x = a + b
                e = jnp.maximum(x, SLOPE * x)
                eec[pl.ds(g * 16, 16)] = jnp.exp(e - mvec)

            pltpu.sync_copy(eec, ee_ref.at[pl.ds(off, CH2B)])
            pltpu.sync_copy(eec, dacc.at[dstv], add=True)

        plsc.subcore_barrier()

        for cc, den_ref in ((0, den0_ref), (1, den1_ref)):
            @pl.when(c == cc)
            def _(den_ref=den_ref):
                pltpu.sync_copy(dacc.at[pl.ds(s * NT, NT)],
                                elv.at[pl.ds(0, NT)])
                pltpu.sync_copy(elv.at[pl.ds(0, NT)],
                                den_ref.at[pl.ds(s * NT, NT)])

    return k(elt, ert, m, srcs, dsts, z1)


def _sc_aggr1(srcs, dsts, ee, f2p, z64):
    @functools.partial(
        pl.kernel,
        out_type=(
            jax.ShapeDtypeStruct((N, L2W), jnp.float32),
            jax.ShapeDtypeStruct((N, L2W), jnp.float32),
        ),
        mesh=_mesh(), **_CP,
        scratch_types=[
            pltpu.VMEM((CH3B,), jnp.int32),
            pltpu.VMEM((CH3B,), jnp.int32),
            pltpu.VMEM((CH3B,), jnp.float32),
            pltpu.VMEM((CH3B, L2W), jnp.float32),
            pltpu.VMEM((CH3B, L2W), jnp.float32),
            pltpu.VMEM_SHARED((N, L2W), jnp.float32),
            pltpu.SemaphoreType.DMA,
        ],
    )
    def k(src_ref, dst_ref, ee_ref, f_ref, z_ref,
          ra_ref, rb_ref, srcv, dstv, eec, rows, scaled, acc, sem):
        c = lax.axis_index("c")
        s = lax.axis_index("s")

        pltpu.sync_copy(z_ref.at[pl.ds(0, 40)], scaled.at[pl.ds(0, 40)])

        @pl.loop(0, NT // 40)
        def _(k5):
            pltpu.sync_copy(scaled.at[pl.ds(0, 40)],
                            acc.at[pl.ds(s * NT + k5 * 40, 40)])
        plsc.subcore_barrier()

        def work(cc, r_ref):
            @pl.loop(0, EPT // CH3B)
            def _(kk):
                off = cc * (E // 2) + s * EPT + kk * CH3B
                pltpu.sync_copy(src_ref.at[pl.ds(off, CH3B)], srcv)
                pltpu.sync_copy(dst_ref.at[pl.ds(off, CH3B)], dstv)
                pltpu.sync_copy(ee_ref.at[pl.ds(off, CH3B)], eec)
                pltpu.async_copy(f_ref.at[srcv], rows, sem).wait()

                @pl.loop(0, CH3B // 16)
                def _(g):
                    ev = eec[pl.ds(g * 16, 16)]
                    for e16 in range(16):
                        eg = g * 16 + e16
                        a = jnp.broadcast_to(ev[e16], (16,))
                        for j in range(L2W // 16):
                            scaled[eg, pl.ds(j * 16, 16)] = (
                                rows[eg, pl.ds(j * 16, 16)] * a)

                pltpu.sync_copy(scaled, acc.at[dstv], add=True)

            plsc.subcore_barrier()

            @pl.loop(0, NT // 40)
            def _(k5):
                pltpu.sync_copy(acc.at[pl.ds(s * NT + k5 * 40, 40)],
                                rows.at[pl.ds(0, 40)])
                pltpu.sync_copy(rows.at[pl.ds(0, 40)],
                                r_ref.at[pl.ds(s * NT + k5 * 40, 40)])

        @pl.when(c == 0)
        def _():
            work(0, ra_ref)

        @pl.when(c == 1)
        def _():
            work(1, rb_ref)

    return k(srcs, dsts, ee, f2p, z64)


def _sc_count(train_nodes, z1):
    @functools.partial(
        pl.kernel,
        out_type=jax.ShapeDtypeStruct((N,), jnp.float32),
        mesh=_mesh(), **_CP,
        scratch_types=[
            pltpu.VMEM((NTRAIN,), jnp.int32),
            pltpu.VMEM((5008,), jnp.float32),
            pltpu.VMEM((N,), jnp.float32),
            pltpu.VMEM_SHARED((N,), jnp.float32),
        ],
    )
    def k(tn_ref, z1_ref, cnt_ref, tnv, ones, zstage, cacc):
        c = lax.axis_index("c")
        s = lax.axis_index("s")

        @pl.when((c == 0) & (s == 0))
        def _():
            pltpu.sync_copy(z1_ref, zstage)
            pltpu.sync_copy(zstage, cacc)
            pltpu.sync_copy(tn_ref, tnv)

            @pl.loop(0, 313)
            def _(i):
                ones[pl.ds(i * 16, 16)] = jnp.ones((16,), jnp.float32)

            pltpu.sync_copy(ones.at[pl.ds(0, NTRAIN)], cacc.at[tnv],
                            add=True)
            pltpu.sync_copy(cacc, zstage)
            pltpu.sync_copy(zstage, cnt_ref)

    return k(train_nodes, z1)


def _blockdiag(a, hh, dh, fw):
    eye = jnp.eye(hh, dtype=a.dtype)
    out = (a[:, None, :] * eye[:, :, None]).reshape(hh, hh * dh)
    if out.shape[1] < fw:
        out = jnp.pad(out, ((0, 0), (0, fw - out.shape[1])))
    return out


def kernel(feats, edge_index, label, train_nodes, W0, al0, ar0, b0,
           W1, al1, ar1, b1, W2, al2, ar2, b2):
    edge = edge_index.astype(jnp.int32)
    srcs = edge[0]
    dsts = edge[1]

    alx0 = _blockdiag(al0, HEADS, D_HEAD, HID)
    arx0 = _blockdiag(ar0, HEADS, D_HEAD, HID)
    alx1 = _blockdiag(al1, HEADS, D_HEAD, HID)
    arx1 = _blockdiag(ar1, HEADS, D_HEAD, HID)
    W2x = jnp.pad(W2, ((0, 0), (0, L2W - OUTC)))
    alx2 = jnp.pad(al2, ((0, 0), (0, L2W - OUTC)))
    arx2 = jnp.pad(ar2, ((0, 0), (0, L2W - OUTC)))
    b2x = jnp.pad(b2, (0, L2W - OUTC)).reshape(1, L2W)
    b0r = b0.reshape(1, HID)
    b1r = b1.reshape(1, HID)

    featsp = jnp.pad(feats, ((0, N - NR), (0, 0)))
    labelp = jnp.pad(label.astype(jnp.int32), (0, N - NR))

    z4 = jnp.zeros((4 * N,), jnp.float32)
    z128 = jnp.zeros((N, 128), jnp.float32)
    z1 = jnp.zeros((N,), jnp.float32)
    z64 = jnp.zeros((N, L2W), jnp.float32)

    fa, fb, ela, elb, era, erb, m = _tc_entry(False, HEADS, HID, D_IN,
                                              (featsp,), W0, alx0, arx0)
    ee0, ee1, d0, d1 = _sc_logits8(ela.reshape(4 * N), elb.reshape(4 * N),
                                   era.reshape(4 * N), erb.reshape(4 * N),
                                   m, srcs, dsts, z4)
    ra, rb = _sc_aggr8(srcs, dsts, ee0, ee1, fa, fb, z128)

    fa, fb, ela, elb, era, erb, m = _tc_entry(True, HEADS, HID, D_IN,
                                              (ra, rb, d0.reshape(N, 4),
                                               d1.reshape(N, 4), b0r),
                                              W1, alx1, arx1)
    ee0, ee1, d0, d1 = _sc_logits8(ela.reshape(4 * N), elb.reshape(4 * N),
                                   era.reshape(4 * N), erb.reshape(4 * N),
                                   m, srcs, dsts, z4)
    ra, rb = _sc_aggr8(srcs, dsts, ee0, ee1, fa, fb, z128)

    f2, elt, ert, m = _tc_entry(True, 1, L2W, D_IN,
                                (ra, rb, d0.reshape(N, 4),
                                 d1.reshape(N, 4), b1r),
                                W2x, alx2, arx2)
    ee, dn0, dn1 = _sc_logits1(elt.reshape(N), ert.reshape(N), m,
                               srcs, dsts, z1)
    ra, rb = _sc_aggr1(srcs, dsts, ee, f2, z64)

    cnt = _sc_count(train_nodes.astype(jnp.int32), z1)

    logp, loss = _tc_final(ra, rb, dn0.reshape(N, 1), dn1.reshape(N, 1),
                           b2x, labelp.reshape(N, 1), cnt.reshape(N, 1))
    return logp[:NR], loss[0, 0]


# speedup vs baseline: 1.6628x; 1.3383x over previous
import functools

import jax
import jax.numpy as jnp
from jax import lax
from jax.experimental import pallas as pl
from jax.experimental.pallas import tpu as pltpu
from jax.experimental.pallas import tpu_sc as plsc

NR = 10000
N = 10240
E = 320000
D_IN = 128
HEADS = 8
D_HEAD = 32
HID = 256
OUTC = 40
SLOPE = 0.2
NTRAIN = 5000

BN = 1024
GRID = N // BN
EPT = E // 32
EPT16 = E // 16

CH2 = 2000
CH3 = 80
CH2B = 2000
CH3B = 80

L2W = 128
NT = N // 16


def _entry_body(use_norm, hh, halfw, *refs):
    if use_norm:
        if hh == 8:
            (ra_ref, rb_ref, da_ref, db_ref, bprev_ref, w_ref, alx_ref,
             arx_ref, fa_ref, fb_ref, *score_refs) = refs
        else:
            (ra_ref, rb_ref, da_ref, db_ref, bprev_ref, w_ref, alx_ref,
             arx_ref, fa_ref, *score_refs) = refs
        den = jnp.concatenate([da_ref[...], db_ref[...]], axis=1)
        rden = jnp.where(den > 0.0, 1.0 / den, 0.0)
        raw = jnp.concatenate([ra_ref[...], rb_ref[...]], axis=1)
        rx = jnp.broadcast_to(rden[:, :, None], (BN, 8, raw.shape[1] // 8))
        h = jnp.maximum(raw * rx.reshape(BN, raw.shape[1]) + bprev_ref[...],
                        0.0)
    else:
        (x_ref, w_ref, alx_ref, arx_ref,
         fa_ref, fb_ref, *score_refs) = refs
        h = x_ref[...]
    i = pl.program_id(0)
    feat = jnp.dot(h, w_ref[...], preferred_element_type=jnp.float32)
    if hh == 8:
        fa_ref[...] = feat[:, :halfw]
        fb_ref[...] = feat[:, halfw:]
    else:
        fa_ref[...] = feat
    el = lax.dot_general(alx_ref[...], feat, (((1,), (1,)), ((), ())),
                         preferred_element_type=jnp.float32)
    er = lax.dot_general(arx_ref[...], feat, (((1,), (1,)), ((), ())),
                         preferred_element_type=jnp.float32)
    if hh == 8:
        ela_ref, elb_ref, era_ref, erb_ref, m_ref, acc_ref = score_refs
        ela_ref[:, pl.ds(i * BN, BN)] = el[:4]
        elb_ref[:, pl.ds(i * BN, BN)] = el[4:]
        era_ref[:, pl.ds(i * BN, BN)] = er[:4]
        erb_ref[:, pl.ds(i * BN, BN)] = er[4:]
    else:
        elt_ref, ert_ref, m_ref, acc_ref = score_refs
        elt_ref[:, pl.ds(i * BN, BN)] = el
        ert_ref[:, pl.ds(i * BN, BN)] = er
    bl = jnp.max(el)
    br = jnp.max(er)

    @pl.when(i == 0)
    def _():
        acc_ref[0] = bl
        acc_ref[1] = br

    @pl.when(i > 0)
    def _():
        acc_ref[0] = jnp.maximum(acc_ref[0], bl)
        acc_ref[1] = jnp.maximum(acc_ref[1], br)

    m = jnp.maximum(acc_ref[0] + acc_ref[1], 0.0)
    m_ref[...] = jnp.full((8, 128), m, jnp.float32)


def _tc_entry(use_norm, hh, fw, din, x_args, w, alx, arx):
    halfw = fw // 2
    in_specs = []
    if use_norm:
        in_specs += [
            pl.BlockSpec((BN, 128), lambda i: (i, 0)),
            pl.BlockSpec((BN, 128), lambda i: (i, 0)),
            pl.BlockSpec((BN, 4), lambda i: (i, 0)),
            pl.BlockSpec((BN, 4), lambda i: (i, 0)),
            pl.BlockSpec((1, 256), lambda i: (0, 0)),
        ]
    else:
        in_specs += [pl.BlockSpec((BN, din), lambda i: (i, 0))]
    in_specs += [
        pl.BlockSpec((din if not use_norm else 256, fw), lambda i: (0, 0)),
        pl.BlockSpec((hh, fw), lambda i: (0, 0)),
        pl.BlockSpec((hh, fw), lambda i: (0, 0)),
    ]
    nsc = 4 if hh == 8 else 2
    nf = 2 if hh == 8 else 1
    fwo = halfw if hh == 8 else fw
    hh2 = hh // 2 if hh == 8 else hh
    out_specs = (
        [pl.BlockSpec((BN, fwo), lambda i: (i, 0))] * nf
        + [pl.BlockSpec((hh2, N), lambda i: (0, 0))] * nsc
        + [pl.BlockSpec((8, 128), lambda i: (0, 0))]
    )
    out_shape = (
        [jax.ShapeDtypeStruct((N, fwo), jnp.float32)] * nf
        + [jax.ShapeDtypeStruct((hh2, N), jnp.float32)] * nsc
        + [jax.ShapeDtypeStruct((8, 128), jnp.float32)]
    )
    fn = pl.pallas_call(
        functools.partial(_entry_body, use_norm, hh, halfw),
        grid=(GRID,),
        in_specs=in_specs,
        out_specs=out_specs,
        out_shape=out_shape,
        scratch_shapes=[pltpu.SMEM((2,), jnp.float32)],
    )
    return fn(*x_args, w, alx, arx)


def _final_body(ra_ref, rb_ref, da_ref, db_ref, b2_ref, lab_ref, cnt_ref,
                logp_ref, loss_ref, acc_ref):
    i = pl.program_id(0)
    raw = ra_ref[...] + rb_ref[...]
    den = da_ref[...] + db_ref[...]
    rden = jnp.where(den > 0.0, 1.0 / den, 0.0)
    h = raw * rden + b2_ref[...]
    colmask = lax.broadcasted_iota(jnp.int32, (1, L2W), 1) < OUTC
    hm = jnp.where(colmask, h, -jnp.inf)
    mx = jnp.max(hm, axis=1, keepdims=True)
    ex = jnp.where(colmask, jnp.exp(h - mx), 0.0)
    lse = jnp.log(jnp.sum(ex, axis=1, keepdims=True)) + mx
    logp = h - lse
    logp_ref[...] = logp[:, :OUTC]
    lab = lab_ref[...]
    onehot = lax.broadcasted_iota(jnp.int32, (BN, L2W), 1) == lab
    pick = jnp.sum(jnp.where(onehot, logp, 0.0), axis=1)
    part = jnp.sum(pick * cnt_ref[...][:, 0])

    @pl.when(i == 0)
    def _():
        acc_ref[0] = part

    @pl.when(i > 0)
    def _():
        acc_ref[0] = acc_ref[0] + part

    loss_ref[...] = jnp.full((1, 1), -acc_ref[0] / float(NTRAIN),
                             jnp.float32)


def _tc_final(ra, rb, d0, d1, b2x, lab2d, cnt2d):
    fn = pl.pallas_call(
        _final_body,
        grid=(GRID,),
        in_specs=[
            pl.BlockSpec((BN, L2W), lambda i: (i, 0)),
            pl.BlockSpec((BN, L2W), lambda i: (i, 0)),
            pl.BlockSpec((BN, 1), lambda i: (i, 0)),
            pl.BlockSpec((BN, 1), lambda i: (i, 0)),
            pl.BlockSpec((1, L2W), lambda i: (0, 0)),
            pl.BlockSpec((BN, 1), lambda i: (i, 0)),
            pl.BlockSpec((BN, 1), lambda i: (i, 0)),
        ],
        out_specs=[
            pl.BlockSpec((BN, OUTC), lambda i: (i, 0)),
            pl.BlockSpec((1, 1), lambda i: (0, 0)),
        ],
        out_shape=[
            jax.ShapeDtypeStruct((N, OUTC), jnp.float32),
            jax.ShapeDtypeStruct((1, 1), jnp.float32),
        ],
        scratch_shapes=[pltpu.SMEM((1,), jnp.float32)],
    )
    return fn(ra, rb, d0, d1, b2x, lab2d, cnt2d)


def _mesh():
    return plsc.VectorSubcoreMesh(core_axis_name="c", subcore_axis_name="s",
                                  num_cores=2, num_subcores=16)


_CP = dict(compiler_params=pltpu.CompilerParams(needs_layout_passes=False))


def _sc_logits8(elaf, elbf, eraf, erbf, m, srcs, dsts, z4):
    @functools.partial(
        pl.kernel,
        out_type=(
            jax.ShapeDtypeStruct((4 * E,), jnp.float32),
            jax.ShapeDtypeStruct((4 * E,), jnp.float32),
            jax.ShapeDtypeStruct((4 * N,), jnp.float32),
            jax.ShapeDtypeStruct((4 * N,), jnp.float32),
        ),
        mesh=_mesh(), **_CP,
        scratch_types=[
            pltpu.VMEM((4 * N,), jnp.float32),
            pltpu.VMEM((4 * N,), jnp.float32),
            pltpu.VMEM((8, 128), jnp.float32),
            pltpu.VMEM((CH2,), jnp.int32),
            pltpu.VMEM((CH2,), jnp.int32),
            pltpu.VMEM((4 * CH2,), jnp.float32),
            pltpu.VMEM((4 * CH2,), jnp.int32),
            pltpu.VMEM_SHARED((4 * N,), jnp.float32),
        ],
    )
    def k(ela_ref, elb_ref, era_ref, erb_ref, m_ref, src_ref, dst_ref, z4_ref,
          ee0_ref, ee1_ref, den0_ref, den1_ref,
          elv, erv, mv, srcv, dstv, eec, didx, dacc):
        c = lax.axis_index("c")
        s = lax.axis_index("s")
        iota = lax.iota(jnp.int32, 16)
        iexp = iota >> 2
        ihead = iota & 3

        pltpu.sync_copy(z4_ref.at[pl.ds(s * 4 * NT, 4 * NT)],
                        elv.at[pl.ds(0, 4 * NT)])
        pltpu.sync_copy(elv.at[pl.ds(0, 4 * NT)],
                        dacc.at[pl.ds(s * 4 * NT, 4 * NT)])
        plsc.subcore_barrier()

        def work(el_in, er_in, ee_ref, den_ref):
            pltpu.sync_copy(el_in, elv)
            pltpu.sync_copy(er_in, erv)
            pltpu.sync_copy(m_ref, mv)
            mvec = mv[0, pl.ds(0, 16)]

            @pl.loop(0, EPT16 // CH2)
            def _(kk):
                off = s * EPT16 + kk * CH2
                pltpu.sync_copy(src_ref.at[pl.ds(off, CH2)], srcv)
                pltpu.sync_copy(dst_ref.at[pl.ds(off, CH2)], dstv)

                @pl.loop(0, CH2 // 16)
                def _(g):
                    for q in range(4):
                        eidx = g * 16 + q * 4 + iexp
                        s4 = plsc.load_gather(srcv, [eidx])
                        d4 = plsc.load_gather(dstv, [eidx])
                        a = plsc.load_gather(elv, [s4 + ihead * N])
                        b = plsc.load_gather(erv, [d4 + ihead * N])
                        x = a + b
                        e = jnp.maximum(x, SLOPE * x)
                        eec[pl.ds(g * 64 + q * 16, 16)] = jnp.exp(e - mvec)
                        didx[pl.ds(g * 64 + q * 16, 16)] = d4 * 4 + ihead

                pltpu.sync_copy(eec, ee_ref.at[pl.ds(off * 4, 4 * CH2)])
                pltpu.sync_copy(eec, dacc.at[didx], add=True)

            plsc.subcore_barrier()
            pltpu.sync_copy(dacc.at[pl.ds(s * 4 * NT, 4 * NT)],
                            elv.at[pl.ds(0, 4 * NT)])
            pltpu.sync_copy(elv.at[pl.ds(0, 4 * NT)],
                            den_ref.at[pl.ds(s * 4 * NT, 4 * NT)])

        @pl.when(c == 0)
        def _():
            work(ela_ref, era_ref, ee0_ref, den0_ref)

        @pl.when(c == 1)
        def _():
            work(elb_ref, erb_ref, ee1_ref, den1_ref)

    return k(elaf, elbf, eraf, erbf, m, srcs, dsts, z4)


def _sc_aggr8(srcs, dsts, ee0, ee1, fa, fb, z128):
    SB = 800
    SUBS = SB // CH3
    NSB = EPT16 // SB
    ROWB = CH3 * 128 * 4

    @functools.partial(
        pl.kernel,
        out_type=(
            jax.ShapeDtypeStruct((N, 128), jnp.float32),
            jax.ShapeDtypeStruct((N, 128), jnp.float32),
        ),
        mesh=_mesh(), **_CP,
        scratch_types=[
            pltpu.VMEM((SB,), jnp.int32),
            pltpu.VMEM((SB,), jnp.int32),
            pltpu.VMEM((SUBS, CH3), jnp.int32),
            pltpu.VMEM((4 * SB,), jnp.float32),
            pltpu.VMEM((CH3, 128), jnp.float32),
            pltpu.VMEM((CH3, 128), jnp.float32),
            pltpu.VMEM((CH3, 128), jnp.float32),
            pltpu.VMEM((CH3, 128), jnp.float32),
            pltpu.VMEM_SHARED((N, 128), jnp.float32),
            pltpu.SemaphoreType.DMA,
            pltpu.SemaphoreType.DMA,
            pltpu.SemaphoreType.DMA,
            pltpu.SemaphoreType.DMA,
        ],
    )
    def k(src_ref, dst_ref, ee0_ref, ee1_ref, fa_ref, fb_ref, z_ref,
          ra_ref, rb_ref, srcv, dstv, dstv2, eec, rows0, rows1, sc0, sc1,
          acc, semg0, semg1, sems0, sems1):
        c = lax.axis_index("c")
        s = lax.axis_index("s")
        rows = (rows0, rows1)
        scaled = (sc0, sc1)
        semg = (semg0, semg1)
        sems = (sems0, sems1)

        pltpu.sync_copy(z_ref.at[pl.ds(0, 40)], sc0.at[pl.ds(0, 40)])

        @pl.loop(0, NT // 40)
        def _(k5):
            pltpu.sync_copy(sc0.at[pl.ds(0, 40)],
                            acc.at[pl.ds(s * NT + k5 * 40, 40)])
        plsc.subcore_barrier()

        def work(ee_ref, f_ref, r_ref):
            @pl.loop(0, NSB)
            def _(kb):
                off = s * EPT16 + kb * SB
                pltpu.sync_copy(src_ref.at[pl.ds(off, SB)], srcv)
                pltpu.sync_copy(dst_ref.at[pl.ds(off, SB)], dstv)
                pltpu.sync_copy(ee_ref.at[pl.ds(off * 4, 4 * SB)], eec)

                @pl.loop(0, SUBS)
                def _(jj):
                    for x in range(CH3 // 16):
                        dstv2[jj, pl.ds(x * 16, 16)] = \
                            dstv[pl.ds(jj * CH3 + x * 16, 16)]

                pltpu.async_copy(f_ref.at[srcv.at[pl.ds(0, CH3)]],
                                 rows0, semg0)

                @pl.loop(0, SUBS // 2)
                def _(jj2):
                    for par in range(2):
                        jj = jj2 * 2 + par

                        @pl.when(jj + 1 < SUBS)
                        def _():
                            pltpu.async_copy(
                                f_ref.at[srcv.at[pl.ds((jj + 1) * CH3, CH3)]],
                                rows[1 - par], semg[1 - par])

                        pltpu.make_async_copy(
                            f_ref.at[srcv.at[pl.ds(0, CH3)]],
                            rows[par], semg[par]).wait()

                        @pl.when(jj >= 2)
                        def _():
                            pltpu.make_async_copy(
                                scaled[par], acc.at[dstv2.at[0]],
                                sems[par]).wait()

                        @pl.loop(0, CH3 // 16)
                        def _(g):
                            base = jj * 4 * CH3 + g * 64
                            ev = [eec[pl.ds(base + q * 16, 16)]
                                  for q in range(4)]
                            for e16 in range(16):
                                q, rr = divmod(e16, 4)
                                eg = g * 16 + e16
                                a = [jnp.broadcast_to(ev[q][4 * rr + h],
                                                      (16,))
                                     for h in range(4)]
                                for jw in range(8):
                                    scaled[par][eg, pl.ds(jw * 16, 16)] = (
                                        rows[par][eg, pl.ds(jw * 16, 16)]
                                        * a[jw // 2])

                        pltpu.async_copy(scaled[par],
                                         acc.at[dstv2.at[jj]],
                                         sems[par], add=True)

                for par in range(2):
                    pltpu.make_async_copy(scaled[par], acc.at[dstv2.at[0]],
                                          sems[par]).wait()

            plsc.subcore_barrier()

            @pl.loop(0, NT // 40)
            def _(k5):
                pltpu.sync_copy(acc.at[pl.ds(s * NT + k5 * 40, 40)],
                                rows0.at[pl.ds(0, 40)])
                pltpu.sync_copy(rows0.at[pl.ds(0, 40)],
                                r_ref.at[pl.ds(s * NT + k5 * 40, 40)])

        @pl.when(c == 0)
        def _():
            work(ee0_ref, fa_ref, ra_ref)

        @pl.when(c == 1)
        def _():
            work(ee1_ref, fb_ref, rb_ref)

    return k(srcs, dsts, ee0, ee1, fa, fb, z128)


def _sc_logits1(elt, ert, m, srcs, dsts, z1):
    @functools.partial(
        pl.kernel,
        out_type=(
            jax.ShapeDtypeStruct((E,), jnp.float32),
            jax.ShapeDtypeStruct((N,), jnp.float32),
            jax.ShapeDtypeStruct((N,), jnp.float32),
        ),
        mesh=_mesh(), **_CP,
        scratch_types=[
            pltpu.VMEM((N,), jnp.float32),
            pltpu.VMEM((N,), jnp.float32),
            pltpu.VMEM((8, 128), jnp.float32),
            pltpu.VMEM((CH2B,), jnp.int32),
            pltpu.VMEM((CH2B,), jnp.int32),
            pltpu.VMEM((CH2B,), jnp.float32),
            pltpu.VMEM_SHARED((N,), jnp.float32),
        ],
    )
    def k(elt_ref, ert_ref, m_ref, src_ref, dst_ref, z1_ref,
          ee_ref, den0_ref, den1_ref,
          elv, erv, mv, srcv, dstv, eec, dacc):
        c = lax.axis_index("c")
        s = lax.axis_index("s")

        pltpu.sync_copy(z1_ref.at[pl.ds(s * NT, NT)], elv.at[pl.ds(0, NT)])
        pltpu.sync_copy(elv.at[pl.ds(0, NT)], dacc.at[pl.ds(s * NT, NT)])
        plsc.subcore_barrier()

        pltpu.sync_copy(elt_ref, elv)
        pltpu.sync_copy(ert_ref, erv)
        pltpu.sync_copy(m_ref, mv)
        mvec = mv[0, pl.ds(0, 16)]

        @pl.loop(0, EPT // CH2B)
        def _(kk):
            off = c * (E // 2) + s * EPT + kk * CH2B
            pltpu.sync_copy(src_ref.at[pl.ds(off, CH2B)], srcv)
            pltpu.sync_copy(dst_ref.at[pl.ds(off, CH2B)], dstv)

            @pl.loop(0, CH2B // 16)
            def _(g):
                s16 = srcv[pl.ds(g * 16, 16)]
                d16 = dstv[pl.ds(g * 16, 16)]
                a = plsc.load_gather(elv, [s16])
                b = plsc.load_gather(erv, [d16])
                x = a + b
                e = jnp.maximum(x, SLOPE * x)
                eec[pl.ds(g * 16, 16)] = jnp.exp(e - mvec)

            pltpu.sync_copy(eec, ee_ref.at[pl.ds(off, CH2B)])
            pltpu.sync_copy(eec, dacc.at[dstv], add=True)

        plsc.subcore_barrier()

        for cc, den_ref in ((0, den0_ref), (1, den1_ref)):
            @pl.when(c == cc)
            def _(den_ref=den_ref):
                pltpu.sync_copy(dacc.at[pl.ds(s * NT, NT)],
                                elv.at[pl.ds(0, NT)])
                pltpu.sync_copy(elv.at[pl.ds(0, NT)],
                                den_ref.at[pl.ds(s * NT, NT)])

    return k(elt, ert, m, srcs, dsts, z1)


def _sc_aggr1(srcs, dsts, ee, f2p, z64):
    @functools.partial(
        pl.kernel,
        out_type=(
            jax.ShapeDtypeStruct((N, L2W), jnp.float32),
            jax.ShapeDtypeStruct((N, L2W), jnp.float32),
        ),
        mesh=_mesh(), **_CP,
        scratch_types=[
            pltpu.VMEM((CH3B,), jnp.int32),
            pltpu.VMEM((CH3B,), jnp.int32),
            pltpu.VMEM((CH3B,), jnp.float32),
            pltpu.VMEM((CH3B, L2W), jnp.float32),
            pltpu.VMEM((CH3B, L2W), jnp.float32),
            pltpu.VMEM_SHARED((N, L2W), jnp.float32),
            pltpu.SemaphoreType.DMA,
        ],
    )
    def k(src_ref, dst_ref, ee_ref, f_ref, z_ref,
          ra_ref, rb_ref, srcv, dstv, eec, rows, scaled, acc, sem):
        c = lax.axis_index("c")
        s = lax.axis_index("s")

        pltpu.sync_copy(z_ref.at[pl.ds(0, 40)], scaled.at[pl.ds(0, 40)])

        @pl.loop(0, NT // 40)
        def _(k5):
            pltpu.sync_copy(scaled.at[pl.ds(0, 40)],
                            acc.at[pl.ds(s * NT + k5 * 40, 40)])
        plsc.subcore_barrier()

        def work(cc, r_ref):
            @pl.loop(0, EPT // CH3B)
            def _(kk):
                off = cc * (E // 2) + s * EPT + kk * CH3B
                pltpu.sync_copy(src_ref.at[pl.ds(off, CH3B)], srcv)
                pltpu.sync_copy(dst_ref.at[pl.ds(off, CH3B)], dstv)
                pltpu.sync_copy(ee_ref.at[pl.ds(off, CH3B)], eec)
                pltpu.async_copy(f_ref.at[srcv], rows, sem).wait()

                @pl.loop(0, CH3B // 16)
                def _(g):
                    ev = eec[pl.ds(g * 16, 16)]
                    for e16 in range(16):
                        eg = g * 16 + e16
                        a = jnp.broadcast_to(ev[e16], (16,))
                        for j in range(L2W // 16):
                            scaled[eg, pl.ds(j * 16, 16)] = (
                                rows[eg, pl.ds(j * 16, 16)] * a)

                pltpu.sync_copy(scaled, acc.at[dstv], add=True)

            plsc.subcore_barrier()

            @pl.loop(0, NT // 40)
            def _(k5):
                pltpu.sync_copy(acc.at[pl.ds(s * NT + k5 * 40, 40)],
                                rows.at[pl.ds(0, 40)])
                pltpu.sync_copy(rows.at[pl.ds(0, 40)],
                                r_ref.at[pl.ds(s * NT + k5 * 40, 40)])

        @pl.when(c == 0)
        def _():
            work(0, ra_ref)

        @pl.when(c == 1)
        def _():
            work(1, rb_ref)

    return k(srcs, dsts, ee, f2p, z64)


def _sc_count(train_nodes, z1):
    @functools.partial(
        pl.kernel,
        out_type=jax.ShapeDtypeStruct((N,), jnp.float32),
        mesh=_mesh(), **_CP,
        scratch_types=[
            pltpu.VMEM((NTRAIN,), jnp.int32),
            pltpu.VMEM((5008,), jnp.float32),
            pltpu.VMEM((N,), jnp.float32),
            pltpu.VMEM_SHARED((N,), jnp.float32),
        ],
    )
    def k(tn_ref, z1_ref, cnt_ref, tnv, ones, zstage, cacc):
        c = lax.axis_index("c")
        s = lax.axis_index("s")

        @pl.when((c == 0) & (s == 0))
        def _():
            pltpu.sync_copy(z1_ref, zstage)
            pltpu.sync_copy(zstage, cacc)
            pltpu.sync_copy(tn_ref, tnv)

            @pl.loop(0, 313)
            def _(i):
                ones[pl.ds(i * 16, 16)] = jnp.ones((16,), jnp.float32)

            pltpu.sync_copy(ones.at[pl.ds(0, NTRAIN)], cacc.at[tnv],
                            add=True)
            pltpu.sync_copy(cacc, zstage)
            pltpu.sync_copy(zstage, cnt_ref)

    return k(train_nodes, z1)


def _blockdiag(a, hh, dh, fw):
    eye = jnp.eye(hh, dtype=a.dtype)
    out = (a[:, None, :] * eye[:, :, None]).reshape(hh, hh * dh)
    if out.shape[1] < fw:
        out = jnp.pad(out, ((0, 0), (0, fw - out.shape[1])))
    return out


def kernel(feats, edge_index, label, train_nodes, W0, al0, ar0, b0,
           W1, al1, ar1, b1, W2, al2, ar2, b2):
    edge = edge_index.astype(jnp.int32)
    srcs = edge[0]
    dsts = edge[1]

    alx0 = _blockdiag(al0, HEADS, D_HEAD, HID)
    arx0 = _blockdiag(ar0, HEADS, D_HEAD, HID)
    alx1 = _blockdiag(al1, HEADS, D_HEAD, HID)
    arx1 = _blockdiag(ar1, HEADS, D_HEAD, HID)
    W2x = jnp.pad(W2, ((0, 0), (0, L2W - OUTC)))
    alx2 = jnp.pad(al2, ((0, 0), (0, L2W - OUTC)))
    arx2 = jnp.pad(ar2, ((0, 0), (0, L2W - OUTC)))
    b2x = jnp.pad(b2, (0, L2W - OUTC)).reshape(1, L2W)
    b0r = b0.reshape(1, HID)
    b1r = b1.reshape(1, HID)

    featsp = jnp.pad(feats, ((0, N - NR), (0, 0)))
    labelp = jnp.pad(label.astype(jnp.int32), (0, N - NR))

    z4 = jnp.zeros((4 * N,), jnp.float32)
    z128 = jnp.zeros((N, 128), jnp.float32)
    z1 = jnp.zeros((N,), jnp.float32)
    z64 = jnp.zeros((N, L2W), jnp.float32)

    fa, fb, ela, elb, era, erb, m = _tc_entry(False, HEADS, HID, D_IN,
                                              (featsp,), W0, alx0, arx0)
    ee0, ee1, d0, d1 = _sc_logits8(ela.reshape(4 * N), elb.reshape(4 * N),
                                   era.reshape(4 * N), erb.reshape(4 * N),
                                   m, srcs, dsts, z4)
    ra, rb = _sc_aggr8(srcs, dsts, ee0, ee1, fa, fb, z128)

    fa, fb, ela, elb, era, erb, m = _tc_entry(True, HEADS, HID, D_IN,
                                              (ra, rb, d0.reshape(N, 4),
                                               d1.reshape(N, 4), b0r),
                                              W1, alx1, arx1)
    ee0, ee1, d0, d1 = _sc_logits8(ela.reshape(4 * N), elb.reshape(4 * N),
                                   era.reshape(4 * N), erb.reshape(4 * N),
                                   m, srcs, dsts, z4)
    ra, rb = _sc_aggr8(srcs, dsts, ee0, ee1, fa, fb, z128)

    f2, elt, ert, m = _tc_entry(True, 1, L2W, D_IN,
                                (ra, rb, d0.reshape(N, 4),
                                 d1.reshape(N, 4), b1r),
                                W2x, alx2, arx2)
    ee, dn0, dn1 = _sc_logits1(elt.reshape(N), ert.reshape(N), m,
                               srcs, dsts, z1)
    ra, rb = _sc_aggr1(srcs, dsts, ee, f2, z64)

    cnt = _sc_count(train_nodes.astype(jnp.int32), z1)

    logp, loss = _tc_final(ra, rb, dn0.reshape(N, 1), dn1.reshape(N, 1),
                           b2x, labelp.reshape(N, 1), cnt.reshape(N, 1))
    return logp[:NR], loss[0, 0]


# speedup vs baseline: 1.9512x; 1.1734x over previous
import functools

import jax
import jax.numpy as jnp
from jax import lax
from jax.experimental import pallas as pl
from jax.experimental.pallas import tpu as pltpu
from jax.experimental.pallas import tpu_sc as plsc

NR = 10000
N = 10240
E = 320000
D_IN = 128
HEADS = 8
D_HEAD = 32
HID = 256
OUTC = 40
SLOPE = 0.2
NTRAIN = 5000

BN = 1024
GRID = N // BN
EPT = E // 32
EPT16 = E // 16

CH2 = 2000
CH3 = 80
CH2B = 2000
CH3B = 80

L2W = 128
NT = N // 16


def _entry_body(use_norm, hh, halfw, *refs):
    if use_norm:
        if hh == 8:
            (ra_ref, rb_ref, da_ref, db_ref, bprev_ref, w_ref, alx_ref,
             arx_ref, fa_ref, fb_ref, *score_refs) = refs
        else:
            (ra_ref, rb_ref, da_ref, db_ref, bprev_ref, w_ref, alx_ref,
             arx_ref, fa_ref, *score_refs) = refs
        den = jnp.concatenate([da_ref[...], db_ref[...]], axis=1)
        rden = jnp.where(den > 0.0, 1.0 / den, 0.0)
        raw = jnp.concatenate([ra_ref[...], rb_ref[...]], axis=1)
        rx = jnp.broadcast_to(rden[:, :, None], (BN, 8, raw.shape[1] // 8))
        h = jnp.maximum(raw * rx.reshape(BN, raw.shape[1]) + bprev_ref[...],
                        0.0)
    else:
        (x_ref, w_ref, alx_ref, arx_ref,
         fa_ref, fb_ref, *score_refs) = refs
        h = x_ref[...]
    i = pl.program_id(0)
    feat = jnp.dot(h, w_ref[...], preferred_element_type=jnp.float32)
    if hh == 8:
        fa_ref[...] = feat[:, :halfw]
        fb_ref[...] = feat[:, halfw:]
    else:
        fa_ref[...] = feat
    el = lax.dot_general(alx_ref[...], feat, (((1,), (1,)), ((), ())),
                         preferred_element_type=jnp.float32)
    er = lax.dot_general(arx_ref[...], feat, (((1,), (1,)), ((), ())),
                         preferred_element_type=jnp.float32)
    if hh == 8:
        ela_ref, elb_ref, era_ref, erb_ref, m_ref, acc_ref = score_refs
        ela_ref[:, pl.ds(i * BN, BN)] = el[:4]
        elb_ref[:, pl.ds(i * BN, BN)] = el[4:]
        era_ref[:, pl.ds(i * BN, BN)] = er[:4]
        erb_ref[:, pl.ds(i * BN, BN)] = er[4:]
    else:
        elt_ref, ert_ref, m_ref, acc_ref = score_refs
        elt_ref[:, pl.ds(i * BN, BN)] = el
        ert_ref[:, pl.ds(i * BN, BN)] = er
    bl = jnp.max(el)
    br = jnp.max(er)

    @pl.when(i == 0)
    def _():
        acc_ref[0] = bl
        acc_ref[1] = br

    @pl.when(i > 0)
    def _():
        acc_ref[0] = jnp.maximum(acc_ref[0], bl)
        acc_ref[1] = jnp.maximum(acc_ref[1], br)

    m = jnp.maximum(acc_ref[0] + acc_ref[1], 0.0)
    m_ref[...] = jnp.full((8, 128), m, jnp.float32)


def _tc_entry(use_norm, hh, fw, din, x_args, w, alx, arx):
    halfw = fw // 2
    in_specs = []
    if use_norm:
        in_specs += [
            pl.BlockSpec((BN, 128), lambda i: (i, 0)),
            pl.BlockSpec((BN, 128), lambda i: (i, 0)),
            pl.BlockSpec((BN, 4), lambda i: (i, 0)),
            pl.BlockSpec((BN, 4), lambda i: (i, 0)),
            pl.BlockSpec((1, 256), lambda i: (0, 0)),
        ]
    else:
        in_specs += [pl.BlockSpec((BN, din), lambda i: (i, 0))]
    in_specs += [
        pl.BlockSpec((din if not use_norm else 256, fw), lambda i: (0, 0)),
        pl.BlockSpec((hh, fw), lambda i: (0, 0)),
        pl.BlockSpec((hh, fw), lambda i: (0, 0)),
    ]
    nsc = 4 if hh == 8 else 2
    nf = 2 if hh == 8 else 1
    fwo = halfw if hh == 8 else fw
    hh2 = hh // 2 if hh == 8 else hh
    out_specs = (
        [pl.BlockSpec((BN, fwo), lambda i: (i, 0))] * nf
        + [pl.BlockSpec((hh2, N), lambda i: (0, 0))] * nsc
        + [pl.BlockSpec((8, 128), lambda i: (0, 0))]
    )
    out_shape = (
        [jax.ShapeDtypeStruct((N, fwo), jnp.float32)] * nf
        + [jax.ShapeDtypeStruct((hh2, N), jnp.float32)] * nsc
        + [jax.ShapeDtypeStruct((8, 128), jnp.float32)]
    )
    fn = pl.pallas_call(
        functools.partial(_entry_body, use_norm, hh, halfw),
        grid=(GRID,),
        in_specs=in_specs,
        out_specs=out_specs,
        out_shape=out_shape,
        scratch_shapes=[pltpu.SMEM((2,), jnp.float32)],
    )
    return fn(*x_args, w, alx, arx)


def _final_body(ra_ref, rb_ref, da_ref, db_ref, b2_ref, lab_ref, cnt_ref,
                logp_ref, loss_ref, acc_ref):
    i = pl.program_id(0)
    raw = ra_ref[...] + rb_ref[...]
    den = da_ref[...] + db_ref[...]
    rden = jnp.where(den > 0.0, 1.0 / den, 0.0)
    h = raw * rden + b2_ref[...]
    colmask = lax.broadcasted_iota(jnp.int32, (1, L2W), 1) < OUTC
    hm = jnp.where(colmask, h, -jnp.inf)
    mx = jnp.max(hm, axis=1, keepdims=True)
    ex = jnp.where(colmask, jnp.exp(h - mx), 0.0)
    lse = jnp.log(jnp.sum(ex, axis=1, keepdims=True)) + mx
    logp = h - lse
    logp_ref[...] = logp[:, :OUTC]
    lab = lab_ref[...]
    onehot = lax.broadcasted_iota(jnp.int32, (BN, L2W), 1) == lab
    pick = jnp.sum(jnp.where(onehot, logp, 0.0), axis=1)
    part = jnp.sum(pick * cnt_ref[...][:, 0])

    @pl.when(i == 0)
    def _():
        acc_ref[0] = part

    @pl.when(i > 0)
    def _():
        acc_ref[0] = acc_ref[0] + part

    loss_ref[...] = jnp.full((1, 1), -acc_ref[0] / float(NTRAIN),
                             jnp.float32)


def _tc_final(ra, rb, d0, d1, b2x, lab2d, cnt2d):
    fn = pl.pallas_call(
        _final_body,
        grid=(GRID,),
        in_specs=[
            pl.BlockSpec((BN, L2W), lambda i: (i, 0)),
            pl.BlockSpec((BN, L2W), lambda i: (i, 0)),
            pl.BlockSpec((BN, 1), lambda i: (i, 0)),
            pl.BlockSpec((BN, 1), lambda i: (i, 0)),
            pl.BlockSpec((1, L2W), lambda i: (0, 0)),
            pl.BlockSpec((BN, 1), lambda i: (i, 0)),
            pl.BlockSpec((BN, 1), lambda i: (i, 0)),
        ],
        out_specs=[
            pl.BlockSpec((BN, OUTC), lambda i: (i, 0)),
            pl.BlockSpec((1, 1), lambda i: (0, 0)),
        ],
        out_shape=[
            jax.ShapeDtypeStruct((N, OUTC), jnp.float32),
            jax.ShapeDtypeStruct((1, 1), jnp.float32),
        ],
        scratch_shapes=[pltpu.SMEM((1,), jnp.float32)],
    )
    return fn(ra, rb, d0, d1, b2x, lab2d, cnt2d)


def _mesh():
    return plsc.VectorSubcoreMesh(core_axis_name="c", subcore_axis_name="s",
                                  num_cores=2, num_subcores=16)


_CP = dict(compiler_params=pltpu.CompilerParams(needs_layout_passes=False))


def _sc_logits8(elaf, elbf, eraf, erbf, m, srcs, dsts, z4):
    @functools.partial(
        pl.kernel,
        out_type=(
            jax.ShapeDtypeStruct((4 * E,), jnp.float32),
            jax.ShapeDtypeStruct((4 * E,), jnp.float32),
            jax.ShapeDtypeStruct((4 * N,), jnp.float32),
            jax.ShapeDtypeStruct((4 * N,), jnp.float32),
        ),
        mesh=_mesh(), **_CP,
        scratch_types=[
            pltpu.VMEM((4 * N,), jnp.float32),
            pltpu.VMEM((4 * N,), jnp.float32),
            pltpu.VMEM((8, 128), jnp.float32),
            pltpu.VMEM((CH2,), jnp.int32),
            pltpu.VMEM((CH2,), jnp.int32),
            pltpu.VMEM((4 * CH2,), jnp.float32),
            pltpu.VMEM((4 * CH2,), jnp.int32),
            pltpu.VMEM_SHARED((4 * N,), jnp.float32),
        ],
    )
    def k(ela_ref, elb_ref, era_ref, erb_ref, m_ref, src_ref, dst_ref, z4_ref,
          ee0_ref, ee1_ref, den0_ref, den1_ref,
          elv, erv, mv, srcv, dstv, eec, didx, dacc):
        c = lax.axis_index("c")
        s = lax.axis_index("s")
        iota = lax.iota(jnp.int32, 16)
        iexp = iota >> 2
        ihead = iota & 3

        pltpu.sync_copy(z4_ref.at[pl.ds(s * 4 * NT, 4 * NT)],
                        elv.at[pl.ds(0, 4 * NT)])
        pltpu.sync_copy(elv.at[pl.ds(0, 4 * NT)],
                        dacc.at[pl.ds(s * 4 * NT, 4 * NT)])
        plsc.subcore_barrier()

        def work(el_in, er_in, ee_ref, den_ref):
            pltpu.sync_copy(el_in, elv)
            pltpu.sync_copy(er_in, erv)
            pltpu.sync_copy(m_ref, mv)
            mvec = mv[0, pl.ds(0, 16)]

            @pl.loop(0, EPT16 // CH2)
            def _(kk):
                off = s * EPT16 + kk * CH2
                pltpu.sync_copy(src_ref.at[pl.ds(off, CH2)], srcv)
                pltpu.sync_copy(dst_ref.at[pl.ds(off, CH2)], dstv)

                @pl.loop(0, CH2 // 16)
                def _(g):
                    for q in range(4):
                        eidx = g * 16 + q * 4 + iexp
                        s4 = plsc.load_gather(srcv, [eidx])
                        d4 = plsc.load_gather(dstv, [eidx])
                        a = plsc.load_gather(elv, [s4 + ihead * N])
                        b = plsc.load_gather(erv, [d4 + ihead * N])
                        x = a + b
                        e = jnp.maximum(x, SLOPE * x)
                        eec[pl.ds(g * 64 + q * 16, 16)] = jnp.exp(e - mvec)
                        didx[pl.ds(g * 64 + q * 16, 16)] = d4 * 4 + ihead

                pltpu.sync_copy(eec, ee_ref.at[pl.ds(off * 4, 4 * CH2)])
                pltpu.sync_copy(eec, dacc.at[didx], add=True)

            plsc.subcore_barrier()
            pltpu.sync_copy(dacc.at[pl.ds(s * 4 * NT, 4 * NT)],
                            elv.at[pl.ds(0, 4 * NT)])
            pltpu.sync_copy(elv.at[pl.ds(0, 4 * NT)],
                            den_ref.at[pl.ds(s * 4 * NT, 4 * NT)])

        @pl.when(c == 0)
        def _():
            work(ela_ref, era_ref, ee0_ref, den0_ref)

        @pl.when(c == 1)
        def _():
            work(elb_ref, erb_ref, ee1_ref, den1_ref)

    return k(elaf, elbf, eraf, erbf, m, srcs, dsts, z4)


def _sc_aggr8(srcs, dsts, ee0, ee1, fa, fb, z128):
    SB = 800
    SUBS = SB // CH3
    NSB = EPT16 // SB
    ROWB = CH3 * 128 * 4

    @functools.partial(
        pl.kernel,
        out_type=(
            jax.ShapeDtypeStruct((N, 128), jnp.float32),
            jax.ShapeDtypeStruct((N, 128), jnp.float32),
        ),
        mesh=_mesh(), **_CP,
        scratch_types=[
            pltpu.VMEM((SB,), jnp.int32),
            pltpu.VMEM((SB,), jnp.int32),
            pltpu.VMEM((SUBS, CH3), jnp.int32),
            pltpu.VMEM((4 * SB,), jnp.float32),
            pltpu.VMEM((CH3, 128), jnp.float32),
            pltpu.VMEM((CH3, 128), jnp.float32),
            pltpu.VMEM((CH3, 128), jnp.float32),
            pltpu.VMEM((CH3, 128), jnp.float32),
            pltpu.VMEM_SHARED((N, 128), jnp.float32),
            pltpu.SemaphoreType.DMA,
            pltpu.SemaphoreType.DMA,
            pltpu.SemaphoreType.DMA,
            pltpu.SemaphoreType.DMA,
        ],
    )
    def k(src_ref, dst_ref, ee0_ref, ee1_ref, fa_ref, fb_ref, z_ref,
          ra_ref, rb_ref, srcv, dstv, dstv2, eec, rows0, rows1, sc0, sc1,
          acc, semg0, semg1, sems0, sems1):
        c = lax.axis_index("c")
        s = lax.axis_index("s")
        rows = (rows0, rows1)
        scaled = (sc0, sc1)
        semg = (semg0, semg1)
        sems = (sems0, sems1)

        pltpu.sync_copy(z_ref.at[pl.ds(0, 40)], sc0.at[pl.ds(0, 40)])

        @pl.loop(0, NT // 40)
        def _(k5):
            pltpu.sync_copy(sc0.at[pl.ds(0, 40)],
                            acc.at[pl.ds(s * NT + k5 * 40, 40)])
        plsc.subcore_barrier()

        def work(ee_ref, f_ref, r_ref):
            @pl.loop(0, NSB)
            def _(kb):
                off = s * EPT16 + kb * SB
                pltpu.sync_copy(src_ref.at[pl.ds(off, SB)], srcv)
                pltpu.sync_copy(dst_ref.at[pl.ds(off, SB)], dstv)
                pltpu.sync_copy(ee_ref.at[pl.ds(off * 4, 4 * SB)], eec)

                @pl.loop(0, SUBS)
                def _(jj):
                    for x in range(CH3 // 16):
                        dstv2[jj, pl.ds(x * 16, 16)] = \
                            dstv[pl.ds(jj * CH3 + x * 16, 16)]

                pltpu.async_copy(f_ref.at[srcv.at[pl.ds(0, CH3)]],
                                 rows0, semg0)

                @pl.loop(0, SUBS // 2)
                def _(jj2):
                    for par in range(2):
                        jj = jj2 * 2 + par

                        @pl.when(jj + 1 < SUBS)
                        def _():
                            pltpu.async_copy(
                                f_ref.at[srcv.at[pl.ds((jj + 1) * CH3, CH3)]],
                                rows[1 - par], semg[1 - par])

                        pltpu.make_async_copy(
                            f_ref.at[srcv.at[pl.ds(0, CH3)]],
                            rows[par], semg[par]).wait()

                        @pl.when(jj >= 2)
                        def _():
                            pltpu.make_async_copy(
                                scaled[par], acc.at[dstv2.at[0]],
                                sems[par]).wait()

                        @pl.loop(0, CH3 // 16)
                        def _(g):
                            base = jj * 4 * CH3 + g * 64
                            ev = [eec[pl.ds(base + q * 16, 16)]
                                  for q in range(4)]
                            for e16 in range(16):
                                q, rr = divmod(e16, 4)
                                eg = g * 16 + e16
                                a = [jnp.broadcast_to(ev[q][4 * rr + h],
                                                      (16,))
                                     for h in range(4)]
                                for jw in range(8):
                                    scaled[par][eg, pl.ds(jw * 16, 16)] = (
                                        rows[par][eg, pl.ds(jw * 16, 16)]
                                        * a[jw // 2])

                        pltpu.async_copy(scaled[par],
                                         acc.at[dstv2.at[jj]],
                                         sems[par], add=True)

                for par in range(2):
                    pltpu.make_async_copy(scaled[par], acc.at[dstv2.at[0]],
                                          sems[par]).wait()

            plsc.subcore_barrier()

            @pl.loop(0, NT // 40)
            def _(k5):
                pltpu.sync_copy(acc.at[pl.ds(s * NT + k5 * 40, 40)],
                                rows0.at[pl.ds(0, 40)])
                pltpu.sync_copy(rows0.at[pl.ds(0, 40)],
                                r_ref.at[pl.ds(s * NT + k5 * 40, 40)])

        @pl.when(c == 0)
        def _():
            work(ee0_ref, fa_ref, ra_ref)

        @pl.when(c == 1)
        def _():
            work(ee1_ref, fb_ref, rb_ref)

    return k(srcs, dsts, ee0, ee1, fa, fb, z128)


def _sc_logits1(elt, ert, m, srcs, dsts, z1):
    @functools.partial(
        pl.kernel,
        out_type=(
            jax.ShapeDtypeStruct((E,), jnp.float32),
            jax.ShapeDtypeStruct((N,), jnp.float32),
            jax.ShapeDtypeStruct((N,), jnp.float32),
        ),
        mesh=_mesh(), **_CP,
        scratch_types=[
            pltpu.VMEM((N,), jnp.float32),
            pltpu.VMEM((N,), jnp.float32),
            pltpu.VMEM((8, 128), jnp.float32),
            pltpu.VMEM((CH2B,), jnp.int32),
            pltpu.VMEM((CH2B,), jnp.int32),
            pltpu.VMEM((CH2B,), jnp.float32),
            pltpu.VMEM_SHARED((N,), jnp.float32),
        ],
    )
    def k(elt_ref, ert_ref, m_ref, src_ref, dst_ref, z1_ref,
          ee_ref, den0_ref, den1_ref,
          elv, erv, mv, srcv, dstv, eec, dacc):
        c = lax.axis_index("c")
        s = lax.axis_index("s")

        pltpu.sync_copy(z1_ref.at[pl.ds(s * NT, NT)], elv.at[pl.ds(0, NT)])
        pltpu.sync_copy(elv.at[pl.ds(0, NT)], dacc.at[pl.ds(s * NT, NT)])
        plsc.subcore_barrier()

        pltpu.sync_copy(elt_ref, elv)
        pltpu.sync_copy(ert_ref, erv)
        pltpu.sync_copy(m_ref, mv)
        mvec = mv[0, pl.ds(0, 16)]

        @pl.loop(0, EPT // CH2B)
        def _(kk):
            off = c * (E // 2) + s * EPT + kk * CH2B
            pltpu.sync_copy(src_ref.at[pl.ds(off, CH2B)], srcv)
            pltpu.sync_copy(dst_ref.at[pl.ds(off, CH2B)], dstv)

            @pl.loop(0, CH2B // 16)
            def _(g):
                s16 = srcv[pl.ds(g * 16, 16)]
                d16 = dstv[pl.ds(g * 16, 16)]
                a = plsc.load_gather(elv, [s16])
                b = plsc.load_gather(erv, [d16])
                x = a + b
                e = jnp.maximum(x, SLOPE * x)
                eec[pl.ds(g * 16, 16)] = jnp.exp(e - mvec)

            pltpu.sync_copy(eec, ee_ref.at[pl.ds(off, CH2B)])
            pltpu.sync_copy(eec, dacc.at[dstv], add=True)

        plsc.subcore_barrier()

        for cc, den_ref in ((0, den0_ref), (1, den1_ref)):
            @pl.when(c == cc)
            def _(den_ref=den_ref):
                pltpu.sync_copy(dacc.at[pl.ds(s * NT, NT)],
                                elv.at[pl.ds(0, NT)])
                pltpu.sync_copy(elv.at[pl.ds(0, NT)],
                                den_ref.at[pl.ds(s * NT, NT)])

    return k(elt, ert, m, srcs, dsts, z1)


def _sc_aggr1(srcs, dsts, ee, f2p, z64):
    SB = 800
    SUBS = SB // CH3B
    NSB = EPT16 // SB

    @functools.partial(
        pl.kernel,
        out_type=(
            jax.ShapeDtypeStruct((N, L2W), jnp.float32),
            jax.ShapeDtypeStruct((N, L2W), jnp.float32),
        ),
        mesh=_mesh(), **_CP,
        scratch_types=[
            pltpu.VMEM((SB,), jnp.int32),
            pltpu.VMEM((SB,), jnp.int32),
            pltpu.VMEM((SUBS, CH3B), jnp.int32),
            pltpu.VMEM((SB,), jnp.float32),
            pltpu.VMEM((CH3B, L2W), jnp.float32),
            pltpu.VMEM((CH3B, L2W), jnp.float32),
            pltpu.VMEM((CH3B, L2W), jnp.float32),
            pltpu.VMEM_SHARED((N, L2W), jnp.float32),
            pltpu.SemaphoreType.DMA,
            pltpu.SemaphoreType.DMA,
        ],
    )
    def k(src_ref, dst_ref, ee_ref, f_ref, z_ref,
          ra_ref, rb_ref, srcv, dstv, dstv2, eec, rows0, rows1, sc0,
          acc, semg0, semg1):
        c = lax.axis_index("c")
        s = lax.axis_index("s")
        rows = (rows0, rows1)
        scaled = (sc0, sc0)
        semg = (semg0, semg1)

        pltpu.sync_copy(z_ref.at[pl.ds(0, 40)], sc0.at[pl.ds(0, 40)])

        @pl.loop(0, NT // 40)
        def _(k5):
            pltpu.sync_copy(sc0.at[pl.ds(0, 40)],
                            acc.at[pl.ds(s * NT + k5 * 40, 40)])
        plsc.subcore_barrier()

        def work(cc, r_ref):
            @pl.loop(0, (NSB + 1 - cc) // 2)
            def _(kb2):
                if True:
                    kb = kb2 * 2 + cc
                    off = s * EPT16 + kb * SB
                    pltpu.sync_copy(src_ref.at[pl.ds(off, SB)], srcv)
                    pltpu.sync_copy(dst_ref.at[pl.ds(off, SB)], dstv)
                    pltpu.sync_copy(ee_ref.at[pl.ds(off, SB)], eec)

                    @pl.loop(0, SUBS)
                    def _(jj):
                        for x in range(CH3B // 16):
                            dstv2[jj, pl.ds(x * 16, 16)] = \
                                dstv[pl.ds(jj * CH3B + x * 16, 16)]

                    pltpu.async_copy(f_ref.at[srcv.at[pl.ds(0, CH3B)]],
                                     rows0, semg0)

                    @pl.loop(0, SUBS // 2)
                    def _(jj2):
                        for par in range(2):
                            jj = jj2 * 2 + par

                            @pl.when(jj + 1 < SUBS)
                            def _():
                                pltpu.async_copy(
                                    f_ref.at[
                                        srcv.at[pl.ds((jj + 1) * CH3B,
                                                      CH3B)]],
                                    rows[1 - par], semg[1 - par])

                            pltpu.make_async_copy(
                                f_ref.at[srcv.at[pl.ds(0, CH3B)]],
                                rows[par], semg[par]).wait()

                            @pl.loop(0, CH3B // 16)
                            def _(g):
                                ev = eec[pl.ds(jj * CH3B + g * 16, 16)]
                                for e16 in range(16):
                                    eg = g * 16 + e16
                                    a = jnp.broadcast_to(ev[e16], (16,))
                                    for jw in range(L2W // 16):
                                        scaled[par][
                                            eg, pl.ds(jw * 16, 16)] = (
                                            rows[par][eg,
                                                      pl.ds(jw * 16, 16)]
                                            * a)

                            pltpu.sync_copy(scaled[par],
                                            acc.at[dstv2.at[jj]],
                                            add=True)

            plsc.subcore_barrier()

            @pl.loop(0, NT // 40)
            def _(k5):
                pltpu.sync_copy(acc.at[pl.ds(s * NT + k5 * 40, 40)],
                                rows0.at[pl.ds(0, 40)])
                pltpu.sync_copy(rows0.at[pl.ds(0, 40)],
                                r_ref.at[pl.ds(s * NT + k5 * 40, 40)])

        @pl.when(c == 0)
        def _():
            work(0, ra_ref)

        @pl.when(c == 1)
        def _():
            work(1, rb_ref)

    return k(srcs, dsts, ee, f2p, z64)


def _sc_count(train_nodes, z1):
    @functools.partial(
        pl.kernel,
        out_type=jax.ShapeDtypeStruct((N,), jnp.float32),
        mesh=_mesh(), **_CP,
        scratch_types=[
            pltpu.VMEM((NTRAIN,), jnp.int32),
            pltpu.VMEM((5008,), jnp.float32),
            pltpu.VMEM((N,), jnp.float32),
            pltpu.VMEM_SHARED((N,), jnp.float32),
        ],
    )
    def k(tn_ref, z1_ref, cnt_ref, tnv, ones, zstage, cacc):
        c = lax.axis_index("c")
        s = lax.axis_index("s")

        @pl.when((c == 0) & (s == 0))
        def _():
            pltpu.sync_copy(z1_ref, zstage)
            pltpu.sync_copy(zstage, cacc)
            pltpu.sync_copy(tn_ref, tnv)

            @pl.loop(0, 313)
            def _(i):
                ones[pl.ds(i * 16, 16)] = jnp.ones((16,), jnp.float32)

            pltpu.sync_copy(ones.at[pl.ds(0, NTRAIN)], cacc.at[tnv],
                            add=True)
            pltpu.sync_copy(cacc, zstage)
            pltpu.sync_copy(zstage, cnt_ref)

    return k(train_nodes, z1)


def _blockdiag(a, hh, dh, fw):
    eye = jnp.eye(hh, dtype=a.dtype)
    out = (a[:, None, :] * eye[:, :, None]).reshape(hh, hh * dh)
    if out.shape[1] < fw:
        out = jnp.pad(out, ((0, 0), (0, fw - out.shape[1])))
    return out


def kernel(feats, edge_index, label, train_nodes, W0, al0, ar0, b0,
           W1, al1, ar1, b1, W2, al2, ar2, b2):
    edge = edge_index.astype(jnp.int32)
    srcs = edge[0]
    dsts = edge[1]

    alx0 = _blockdiag(al0, HEADS, D_HEAD, HID)
    arx0 = _blockdiag(ar0, HEADS, D_HEAD, HID)
    alx1 = _blockdiag(al1, HEADS, D_HEAD, HID)
    arx1 = _blockdiag(ar1, HEADS, D_HEAD, HID)
    W2x = jnp.pad(W2, ((0, 0), (0, L2W - OUTC)))
    alx2 = jnp.pad(al2, ((0, 0), (0, L2W - OUTC)))
    arx2 = jnp.pad(ar2, ((0, 0), (0, L2W - OUTC)))
    b2x = jnp.pad(b2, (0, L2W - OUTC)).reshape(1, L2W)
    b0r = b0.reshape(1, HID)
    b1r = b1.reshape(1, HID)

    featsp = jnp.pad(feats, ((0, N - NR), (0, 0)))
    labelp = jnp.pad(label.astype(jnp.int32), (0, N - NR))

    z4 = jnp.zeros((4 * N,), jnp.float32)
    z128 = jnp.zeros((N, 128), jnp.float32)
    z1 = jnp.zeros((N,), jnp.float32)
    z64 = jnp.zeros((N, L2W), jnp.float32)

    fa, fb, ela, elb, era, erb, m = _tc_entry(False, HEADS, HID, D_IN,
                                              (featsp,), W0, alx0, arx0)
    ee0, ee1, d0, d1 = _sc_logits8(ela.reshape(4 * N), elb.reshape(4 * N),
                                   era.reshape(4 * N), erb.reshape(4 * N),
                                   m, srcs, dsts, z4)
    ra, rb = _sc_aggr8(srcs, dsts, ee0, ee1, fa, fb, z128)

    fa, fb, ela, elb, era, erb, m = _tc_entry(True, HEADS, HID, D_IN,
                                              (ra, rb, d0.reshape(N, 4),
                                               d1.reshape(N, 4), b0r),
                                              W1, alx1, arx1)
    ee0, ee1, d0, d1 = _sc_logits8(ela.reshape(4 * N), elb.reshape(4 * N),
                                   era.reshape(4 * N), erb.reshape(4 * N),
                                   m, srcs, dsts, z4)
    ra, rb = _sc_aggr8(srcs, dsts, ee0, ee1, fa, fb, z128)

    f2, elt, ert, m = _tc_entry(True, 1, L2W, D_IN,
                                (ra, rb, d0.reshape(N, 4),
                                 d1.reshape(N, 4), b1r),
                                W2x, alx2, arx2)
    ee, dn0, dn1 = _sc_logits1(elt.reshape(N), ert.reshape(N), m,
                               srcs, dsts, z1)
    ra, rb = _sc_aggr1(srcs, dsts, ee, f2, z64)

    cnt = _sc_count(train_nodes.astype(jnp.int32), z1)

    logp, loss = _tc_final(ra, rb, dn0.reshape(N, 1), dn1.reshape(N, 1),
                           b2x, labelp.reshape(N, 1), cnt.reshape(N, 1))
    return logp[:NR], loss[0, 0]


# speedup vs baseline: 2.0613x; 1.0564x over previous
import functools

import jax
import jax.numpy as jnp
from jax import lax
from jax.experimental import pallas as pl
from jax.experimental.pallas import tpu as pltpu
from jax.experimental.pallas import tpu_sc as plsc

NR = 10000
N = 10240
E = 320000
D_IN = 128
HEADS = 8
D_HEAD = 32
HID = 256
OUTC = 40
SLOPE = 0.2
NTRAIN = 5000

BN = 1024
GRID = N // BN
EPT = E // 32
EPT16 = E // 16

CH2 = 2000
CH3 = 80
CH2B = 2000
CH3B = 80

L2W = 128
NT = N // 16


def _entry_body(use_norm, hh, halfw, *refs):
    if use_norm:
        if hh == 8:
            (ra_ref, rb_ref, da_ref, db_ref, bprev_ref, w_ref, alx_ref,
             arx_ref, fa_ref, fb_ref, *score_refs) = refs
        else:
            (ra_ref, rb_ref, da_ref, db_ref, bprev_ref, w_ref, alx_ref,
             arx_ref, fa_ref, *score_refs) = refs
        den = jnp.concatenate([da_ref[...], db_ref[...]], axis=1)
        rden = jnp.where(den > 0.0, 1.0 / den, 0.0)
        raw = jnp.concatenate([ra_ref[...], rb_ref[...]], axis=1)
        rx = jnp.broadcast_to(rden[:, :, None], (BN, 8, raw.shape[1] // 8))
        h = jnp.maximum(raw * rx.reshape(BN, raw.shape[1]) + bprev_ref[...],
                        0.0)
    else:
        (x_ref, w_ref, alx_ref, arx_ref,
         fa_ref, fb_ref, *score_refs) = refs
        h = x_ref[...]
    i = pl.program_id(0)
    feat = jnp.dot(h, w_ref[...], preferred_element_type=jnp.float32)
    if hh == 8:
        fa_ref[...] = feat[:, :halfw]
        fb_ref[...] = feat[:, halfw:]
    else:
        fa_ref[...] = feat
    el = lax.dot_general(alx_ref[...], feat, (((1,), (1,)), ((), ())),
                         preferred_element_type=jnp.float32)
    er = lax.dot_general(arx_ref[...], feat, (((1,), (1,)), ((), ())),
                         preferred_element_type=jnp.float32)
    if hh == 8:
        ela_ref, elb_ref, era_ref, erb_ref, m_ref, acc_ref = score_refs
        ela_ref[:, pl.ds(i * BN, BN)] = el[:4]
        elb_ref[:, pl.ds(i * BN, BN)] = el[4:]
        era_ref[:, pl.ds(i * BN, BN)] = er[:4]
        erb_ref[:, pl.ds(i * BN, BN)] = er[4:]
    else:
        elt_ref, ert_ref, m_ref, acc_ref = score_refs
        elt_ref[:, pl.ds(i * BN, BN)] = el
        ert_ref[:, pl.ds(i * BN, BN)] = er
    bl = jnp.max(el)
    br = jnp.max(er)

    @pl.when(i == 0)
    def _():
        acc_ref[0] = bl
        acc_ref[1] = br

    @pl.when(i > 0)
    def _():
        acc_ref[0] = jnp.maximum(acc_ref[0], bl)
        acc_ref[1] = jnp.maximum(acc_ref[1], br)

    m = jnp.maximum(acc_ref[0] + acc_ref[1], 0.0)
    m_ref[...] = jnp.full((8, 128), m, jnp.float32)


def _tc_entry(use_norm, hh, fw, din, x_args, w, alx, arx):
    halfw = fw // 2
    in_specs = []
    if use_norm:
        in_specs += [
            pl.BlockSpec((BN, 128), lambda i: (i, 0)),
            pl.BlockSpec((BN, 128), lambda i: (i, 0)),
            pl.BlockSpec((BN, 4), lambda i: (i, 0)),
            pl.BlockSpec((BN, 4), lambda i: (i, 0)),
            pl.BlockSpec((1, 256), lambda i: (0, 0)),
        ]
    else:
        in_specs += [pl.BlockSpec((BN, din), lambda i: (i, 0))]
    in_specs += [
        pl.BlockSpec((din if not use_norm else 256, fw), lambda i: (0, 0)),
        pl.BlockSpec((hh, fw), lambda i: (0, 0)),
        pl.BlockSpec((hh, fw), lambda i: (0, 0)),
    ]
    nsc = 4 if hh == 8 else 2
    nf = 2 if hh == 8 else 1
    fwo = halfw if hh == 8 else fw
    hh2 = hh // 2 if hh == 8 else hh
    out_specs = (
        [pl.BlockSpec((BN, fwo), lambda i: (i, 0))] * nf
        + [pl.BlockSpec((hh2, N), lambda i: (0, 0))] * nsc
        + [pl.BlockSpec((8, 128), lambda i: (0, 0))]
    )
    out_shape = (
        [jax.ShapeDtypeStruct((N, fwo), jnp.float32)] * nf
        + [jax.ShapeDtypeStruct((hh2, N), jnp.float32)] * nsc
        + [jax.ShapeDtypeStruct((8, 128), jnp.float32)]
    )
    fn = pl.pallas_call(
        functools.partial(_entry_body, use_norm, hh, halfw),
        grid=(GRID,),
        in_specs=in_specs,
        out_specs=out_specs,
        out_shape=out_shape,
        scratch_shapes=[pltpu.SMEM((2,), jnp.float32)],
    )
    return fn(*x_args, w, alx, arx)


def _final_body(ra_ref, rb_ref, da_ref, db_ref, b2_ref, lab_ref, cnt_ref,
                logp_ref, loss_ref, acc_ref):
    i = pl.program_id(0)
    raw = ra_ref[...] + rb_ref[...]
    den = da_ref[...] + db_ref[...]
    rden = jnp.where(den > 0.0, 1.0 / den, 0.0)
    h = raw * rden + b2_ref[...]
    colmask = lax.broadcasted_iota(jnp.int32, (1, L2W), 1) < OUTC
    hm = jnp.where(colmask, h, -jnp.inf)
    mx = jnp.max(hm, axis=1, keepdims=True)
    ex = jnp.where(colmask, jnp.exp(h - mx), 0.0)
    lse = jnp.log(jnp.sum(ex, axis=1, keepdims=True)) + mx
    logp = h - lse
    logp_ref[...] = logp[:, :OUTC]
    lab = lab_ref[...]
    onehot = lax.broadcasted_iota(jnp.int32, (BN, L2W), 1) == lab
    pick = jnp.sum(jnp.where(onehot, logp, 0.0), axis=1)
    part = jnp.sum(pick * cnt_ref[...][:, 0])

    @pl.when(i == 0)
    def _():
        acc_ref[0] = part

    @pl.when(i > 0)
    def _():
        acc_ref[0] = acc_ref[0] + part

    loss_ref[...] = jnp.full((1, 1), -acc_ref[0] / float(NTRAIN),
                             jnp.float32)


def _tc_final(ra, rb, d0, d1, b2x, lab2d, cnt2d):
    fn = pl.pallas_call(
        _final_body,
        grid=(GRID,),
        in_specs=[
            pl.BlockSpec((BN, L2W), lambda i: (i, 0)),
            pl.BlockSpec((BN, L2W), lambda i: (i, 0)),
            pl.BlockSpec((BN, 1), lambda i: (i, 0)),
            pl.BlockSpec((BN, 1), lambda i: (i, 0)),
            pl.BlockSpec((1, L2W), lambda i: (0, 0)),
            pl.BlockSpec((BN, 1), lambda i: (i, 0)),
            pl.BlockSpec((BN, 1), lambda i: (i, 0)),
        ],
        out_specs=[
            pl.BlockSpec((BN, OUTC), lambda i: (i, 0)),
            pl.BlockSpec((1, 1), lambda i: (0, 0)),
        ],
        out_shape=[
            jax.ShapeDtypeStruct((N, OUTC), jnp.float32),
            jax.ShapeDtypeStruct((1, 1), jnp.float32),
        ],
        scratch_shapes=[pltpu.SMEM((1,), jnp.float32)],
    )
    return fn(ra, rb, d0, d1, b2x, lab2d, cnt2d)


def _mesh():
    return plsc.VectorSubcoreMesh(core_axis_name="c", subcore_axis_name="s",
                                  num_cores=2, num_subcores=16)


_CP = dict(compiler_params=pltpu.CompilerParams(needs_layout_passes=False))


def _sc_logits8(elaf, elbf, eraf, erbf, m, srcs, dsts, z4):
    NCH = EPT16 // CH2

    @functools.partial(
        pl.kernel,
        out_type=(
            jax.ShapeDtypeStruct((4 * E,), jnp.float32),
            jax.ShapeDtypeStruct((4 * E,), jnp.float32),
            jax.ShapeDtypeStruct((4 * N,), jnp.float32),
            jax.ShapeDtypeStruct((4 * N,), jnp.float32),
        ),
        mesh=_mesh(), **_CP,
        scratch_types=[
            pltpu.VMEM((4 * N,), jnp.float32),
            pltpu.VMEM((4 * N,), jnp.float32),
            pltpu.VMEM((8, 128), jnp.float32),
            pltpu.VMEM((CH2,), jnp.int32),
            pltpu.VMEM((CH2,), jnp.int32),
            pltpu.VMEM((4 * CH2,), jnp.float32),
            pltpu.VMEM((4 * CH2,), jnp.int32),
            pltpu.VMEM((CH2,), jnp.int32),
            pltpu.VMEM((CH2,), jnp.int32),
            pltpu.VMEM((4 * CH2,), jnp.float32),
            pltpu.VMEM((4 * CH2,), jnp.int32),
            pltpu.VMEM_SHARED((4 * N,), jnp.float32),
            pltpu.SemaphoreType.DMA,
            pltpu.SemaphoreType.DMA,
            pltpu.SemaphoreType.DMA,
            pltpu.SemaphoreType.DMA,
            pltpu.SemaphoreType.DMA,
            pltpu.SemaphoreType.DMA,
        ],
    )
    def k(ela_ref, elb_ref, era_ref, erb_ref, m_ref, src_ref, dst_ref, z4_ref,
          ee0_ref, ee1_ref, den0_ref, den1_ref,
          elv, erv, mv, srcv0, dstv0, eec0, didx0, srcv1, dstv1, eec1,
          didx1, dacc, seml0, seml1, sems0, sems1, sema0, sema1):
        c = lax.axis_index("c")
        s = lax.axis_index("s")
        iota = lax.iota(jnp.int32, 16)
        iexp = iota >> 2
        ihead = iota & 3
        srcv = (srcv0, srcv1)
        dstv = (dstv0, dstv1)
        eec = (eec0, eec1)
        didx = (didx0, didx1)
        seml = (seml0, seml1)
        sems = (sems0, sems1)
        sema = (sema0, sema1)

        pltpu.sync_copy(z4_ref.at[pl.ds(s * 4 * NT, 4 * NT)],
                        elv.at[pl.ds(0, 4 * NT)])
        pltpu.sync_copy(elv.at[pl.ds(0, 4 * NT)],
                        dacc.at[pl.ds(s * 4 * NT, 4 * NT)])
        plsc.subcore_barrier()

        def work(el_in, er_in, ee_ref, den_ref):
            pltpu.sync_copy(el_in, elv)
            pltpu.sync_copy(er_in, erv)
            pltpu.sync_copy(m_ref, mv)
            mvec = mv[0, pl.ds(0, 16)]

            def load(kchunk, par):
                off = s * EPT16 + kchunk * CH2
                pltpu.async_copy(src_ref.at[pl.ds(off, CH2)], srcv[par],
                                 seml[par])
                pltpu.async_copy(dst_ref.at[pl.ds(off, CH2)], dstv[par],
                                 seml[par])

            def consume(kchunk, par):
                off = s * EPT16 + kchunk * CH2
                pltpu.make_async_copy(src_ref.at[pl.ds(0, CH2)], srcv[par],
                                      seml[par]).wait()
                pltpu.make_async_copy(dst_ref.at[pl.ds(0, CH2)], dstv[par],
                                      seml[par]).wait()

                @pl.when(kchunk >= 2)
                def _():
                    pltpu.make_async_copy(
                        eec[par], ee_ref.at[pl.ds(0, 4 * CH2)],
                        sems[par]).wait()
                    pltpu.make_async_copy(
                        eec[par], dacc.at[didx[par]], sema[par]).wait()

                @pl.loop(0, CH2 // 16)
                def _(g):
                    for q in range(4):
                        eidx = g * 16 + q * 4 + iexp
                        s4 = plsc.load_gather(srcv[par], [eidx])
                        d4 = plsc.load_gather(dstv[par], [eidx])
                        a = plsc.load_gather(elv, [s4 + ihead * N])
                        b = plsc.load_gather(erv, [d4 + ihead * N])
                        x = a + b
                        e = jnp.maximum(x, SLOPE * x)
                        eec[par][pl.ds(g * 64 + q * 16, 16)] = \
                            jnp.exp(e - mvec)
                        didx[par][pl.ds(g * 64 + q * 16, 16)] = \
                            d4 * 4 + ihead

                pltpu.async_copy(eec[par], ee_ref.at[pl.ds(off * 4, 4 * CH2)],
                                 sems[par])
                pltpu.async_copy(eec[par], dacc.at[didx[par]], sema[par],
                                 add=True)

            load(0, 0)

            @pl.loop(0, NCH // 2)
            def _(kk2):
                for par in range(2):
                    kchunk = kk2 * 2 + par

                    @pl.when(kchunk + 1 < NCH)
                    def _():
                        load(kchunk + 1, 1 - par)

                    consume(kchunk, par)

            for par in range(2):
                pltpu.make_async_copy(eec[par], ee_ref.at[pl.ds(0, 4 * CH2)],
                                      sems[par]).wait()
                pltpu.make_async_copy(eec[par], dacc.at[didx[par]],
                                      sema[par]).wait()

            plsc.subcore_barrier()
            pltpu.sync_copy(dacc.at[pl.ds(s * 4 * NT, 4 * NT)],
                            elv.at[pl.ds(0, 4 * NT)])
            pltpu.sync_copy(elv.at[pl.ds(0, 4 * NT)],
                            den_ref.at[pl.ds(s * 4 * NT, 4 * NT)])

        @pl.when(c == 0)
        def _():
            work(ela_ref, era_ref, ee0_ref, den0_ref)

        @pl.when(c == 1)
        def _():
            work(elb_ref, erb_ref, ee1_ref, den1_ref)

    return k(elaf, elbf, eraf, erbf, m, srcs, dsts, z4)


def _sc_aggr8(srcs, dsts, ee0, ee1, fa, fb, z128):
    SB = 800
    SUBS = SB // CH3
    NSB = EPT16 // SB
    ROWB = CH3 * 128 * 4

    @functools.partial(
        pl.kernel,
        out_type=(
            jax.ShapeDtypeStruct((N, 128), jnp.float32),
            jax.ShapeDtypeStruct((N, 128), jnp.float32),
        ),
        mesh=_mesh(), **_CP,
        scratch_types=[
            pltpu.VMEM((SB,), jnp.int32),
            pltpu.VMEM((SB,), jnp.int32),
            pltpu.VMEM((SUBS, CH3), jnp.int32),
            pltpu.VMEM((4 * SB,), jnp.float32),
            pltpu.VMEM((CH3, 128), jnp.float32),
            pltpu.VMEM((CH3, 128), jnp.float32),
            pltpu.VMEM((CH3, 128), jnp.float32),
            pltpu.VMEM((CH3, 128), jnp.float32),
            pltpu.VMEM_SHARED((N, 128), jnp.float32),
            pltpu.SemaphoreType.DMA,
            pltpu.SemaphoreType.DMA,
            pltpu.SemaphoreType.DMA,
            pltpu.SemaphoreType.DMA,
        ],
    )
    def k(src_ref, dst_ref, ee0_ref, ee1_ref, fa_ref, fb_ref, z_ref,
          ra_ref, rb_ref, srcv, dstv, dstv2, eec, rows0, rows1, sc0, sc1,
          acc, semg0, semg1, sems0, sems1):
        c = lax.axis_index("c")
        s = lax.axis_index("s")
        rows = (rows0, rows1)
        scaled = (sc0, sc1)
        semg = (semg0, semg1)
        sems = (sems0, sems1)

        pltpu.sync_copy(z_ref.at[pl.ds(0, 40)], sc0.at[pl.ds(0, 40)])

        @pl.loop(0, NT // 40)
        def _(k5):
            pltpu.sync_copy(sc0.at[pl.ds(0, 40)],
                            acc.at[pl.ds(s * NT + k5 * 40, 40)])
        plsc.subcore_barrier()

        def work(ee_ref, f_ref, r_ref):
            @pl.loop(0, NSB)
            def _(kb):
                off = s * EPT16 + kb * SB
                pltpu.sync_copy(src_ref.at[pl.ds(off, SB)], srcv)
                pltpu.sync_copy(dst_ref.at[pl.ds(off, SB)], dstv)
                pltpu.sync_copy(ee_ref.at[pl.ds(off * 4, 4 * SB)], eec)

                @pl.loop(0, SUBS)
                def _(jj):
                    for x in range(CH3 // 16):
                        dstv2[jj, pl.ds(x * 16, 16)] = \
                            dstv[pl.ds(jj * CH3 + x * 16, 16)]

                pltpu.async_copy(f_ref.at[srcv.at[pl.ds(0, CH3)]],
                                 rows0, semg0)

                @pl.loop(0, SUBS // 2)
                def _(jj2):
                    for par in range(2):
                        jj = jj2 * 2 + par

                        @pl.when(jj + 1 < SUBS)
                        def _():
                            pltpu.async_copy(
                                f_ref.at[srcv.at[pl.ds((jj + 1) * CH3, CH3)]],
                                rows[1 - par], semg[1 - par])

                        pltpu.make_async_copy(
                            f_ref.at[srcv.at[pl.ds(0, CH3)]],
                            rows[par], semg[par]).wait()

                        @pl.when(jj >= 2)
                        def _():
                            pltpu.make_async_copy(
                                scaled[par], acc.at[dstv2.at[0]],
                                sems[par]).wait()

                        @pl.loop(0, CH3 // 16)
                        def _(g):
                            base = jj * 4 * CH3 + g * 64
                            ev = [eec[pl.ds(base + q * 16, 16)]
                                  for q in range(4)]
                            for e16 in range(16):
                                q, rr = divmod(e16, 4)
                                eg = g * 16 + e16
                                a = [jnp.broadcast_to(ev[q][4 * rr + h],
                                                      (16,))
                                     for h in range(4)]
                                for jw in range(8):
                                    scaled[par][eg, pl.ds(jw * 16, 16)] = (
                                        rows[par][eg, pl.ds(jw * 16, 16)]
                                        * a[jw // 2])

                        pltpu.async_copy(scaled[par],
                                         acc.at[dstv2.at[jj]],
                                         sems[par], add=True)

                for par in range(2):
                    pltpu.make_async_copy(scaled[par], acc.at[dstv2.at[0]],
                                          sems[par]).wait()

            plsc.subcore_barrier()

            @pl.loop(0, NT // 40)
            def _(k5):
                pltpu.sync_copy(acc.at[pl.ds(s * NT + k5 * 40, 40)],
                                rows0.at[pl.ds(0, 40)])
                pltpu.sync_copy(rows0.at[pl.ds(0, 40)],
                                r_ref.at[pl.ds(s * NT + k5 * 40, 40)])

        @pl.when(c == 0)
        def _():
            work(ee0_ref, fa_ref, ra_ref)

        @pl.when(c == 1)
        def _():
            work(ee1_ref, fb_ref, rb_ref)

    return k(srcs, dsts, ee0, ee1, fa, fb, z128)


def _sc_logits1(elt, ert, m, srcs, dsts, z1):
    @functools.partial(
        pl.kernel,
        out_type=(
            jax.ShapeDtypeStruct((E,), jnp.float32),
            jax.ShapeDtypeStruct((N,), jnp.float32),
            jax.ShapeDtypeStruct((N,), jnp.float32),
        ),
        mesh=_mesh(), **_CP,
        scratch_types=[
            pltpu.VMEM((N,), jnp.float32),
            pltpu.VMEM((N,), jnp.float32),
            pltpu.VMEM((8, 128), jnp.float32),
            pltpu.VMEM((CH2B,), jnp.int32),
            pltpu.VMEM((CH2B,), jnp.int32),
            pltpu.VMEM((CH2B,), jnp.float32),
            pltpu.VMEM_SHARED((N,), jnp.float32),
        ],
    )
    def k(elt_ref, ert_ref, m_ref, src_ref, dst_ref, z1_ref,
          ee_ref, den0_ref, den1_ref,
          elv, erv, mv, srcv, dstv, eec, dacc):
        c = lax.axis_index("c")
        s = lax.axis_index("s")

        pltpu.sync_copy(z1_ref.at[pl.ds(s * NT, NT)], elv.at[pl.ds(0, NT)])
        pltpu.sync_copy(elv.at[pl.ds(0, NT)], dacc.at[pl.ds(s * NT, NT)])
        plsc.subcore_barrier()

        pltpu.sync_copy(elt_ref, elv)
        pltpu.sync_copy(ert_ref, erv)
        pltpu.sync_copy(m_ref, mv)
        mvec = mv[0, pl.ds(0, 16)]

        @pl.loop(0, EPT // CH2B)
        def _(kk):
            off = c * (E // 2) + s * EPT + kk * CH2B
            pltpu.sync_copy(src_ref.at[pl.ds(off, CH2B)], srcv)
            pltpu.sync_copy(dst_ref.at[pl.ds(off, CH2B)], dstv)

            @pl.loop(0, CH2B // 16)
            def _(g):
                s16 = srcv[pl.ds(g * 16, 16)]
                d16 = dstv[pl.ds(g * 16, 16)]
                a = plsc.load_gather(elv, [s16])
                b = plsc.load_gather(erv, [d16])
                x = a + b
                e = jnp.maximum(x, SLOPE * x)
                eec[pl.ds(g * 16, 16)] = jnp.exp(e - mvec)

            pltpu.sync_copy(eec, ee_ref.at[pl.ds(off, CH2B)])
            pltpu.sync_copy(eec, dacc.at[dstv], add=True)

        plsc.subcore_barrier()

        for cc, den_ref in ((0, den0_ref), (1, den1_ref)):
            @pl.when(c == cc)
            def _(den_ref=den_ref):
                pltpu.sync_copy(dacc.at[pl.ds(s * NT, NT)],
                                elv.at[pl.ds(0, NT)])
                pltpu.sync_copy(elv.at[pl.ds(0, NT)],
                                den_ref.at[pl.ds(s * NT, NT)])

    return k(elt, ert, m, srcs, dsts, z1)


def _sc_aggr1(srcs, dsts, ee, f2p, z64):
    SB = 800
    SUBS = SB // CH3B
    NSB = EPT16 // SB

    @functools.partial(
        pl.kernel,
        out_type=(
            jax.ShapeDtypeStruct((N, L2W), jnp.float32),
            jax.ShapeDtypeStruct((N, L2W), jnp.float32),
        ),
        mesh=_mesh(), **_CP,
        scratch_types=[
            pltpu.VMEM((SB,), jnp.int32),
            pltpu.VMEM((SB,), jnp.int32),
            pltpu.VMEM((SUBS, CH3B), jnp.int32),
            pltpu.VMEM((SB,), jnp.float32),
            pltpu.VMEM((CH3B, L2W), jnp.float32),
            pltpu.VMEM((CH3B, L2W), jnp.float32),
            pltpu.VMEM((CH3B, L2W), jnp.float32),
            pltpu.VMEM_SHARED((N, L2W), jnp.float32),
            pltpu.SemaphoreType.DMA,
            pltpu.SemaphoreType.DMA,
        ],
    )
    def k(src_ref, dst_ref, ee_ref, f_ref, z_ref,
          ra_ref, rb_ref, srcv, dstv, dstv2, eec, rows0, rows1, sc0,
          acc, semg0, semg1):
        c = lax.axis_index("c")
        s = lax.axis_index("s")
        rows = (rows0, rows1)
        scaled = (sc0, sc0)
        semg = (semg0, semg1)

        pltpu.sync_copy(z_ref.at[pl.ds(0, 40)], sc0.at[pl.ds(0, 40)])

        @pl.loop(0, NT // 40)
        def _(k5):
            pltpu.sync_copy(sc0.at[pl.ds(0, 40)],
                            acc.at[pl.ds(s * NT + k5 * 40, 40)])
        plsc.subcore_barrier()

        def work(cc, r_ref):
            @pl.loop(0, (NSB + 1 - cc) // 2)
            def _(kb2):
                if True:
                    kb = kb2 * 2 + cc
                    off = s * EPT16 + kb * SB
                    pltpu.sync_copy(src_ref.at[pl.ds(off, SB)], srcv)
                    pltpu.sync_copy(dst_ref.at[pl.ds(off, SB)], dstv)
                    pltpu.sync_copy(ee_ref.at[pl.ds(off, SB)], eec)

                    @pl.loop(0, SUBS)
                    def _(jj):
                        for x in range(CH3B // 16):
                            dstv2[jj, pl.ds(x * 16, 16)] = \
                                dstv[pl.ds(jj * CH3B + x * 16, 16)]

                    pltpu.async_copy(f_ref.at[srcv.at[pl.ds(0, CH3B)]],
                                     rows0, semg0)

                    @pl.loop(0, SUBS // 2)
                    def _(jj2):
                        for par in range(2):
                            jj = jj2 * 2 + par

                            @pl.when(jj + 1 < SUBS)
                            def _():
                                pltpu.async_copy(
                                    f_ref.at[
                                        srcv.at[pl.ds((jj + 1) * CH3B,
                                                      CH3B)]],
                                    rows[1 - par], semg[1 - par])

                            pltpu.make_async_copy(
                                f_ref.at[srcv.at[pl.ds(0, CH3B)]],
                                rows[par], semg[par]).wait()

                            @pl.loop(0, CH3B // 16)
                            def _(g):
                                ev = eec[pl.ds(jj * CH3B + g * 16, 16)]
                                for e16 in range(16):
                                    eg = g * 16 + e16
                                    a = jnp.broadcast_to(ev[e16], (16,))
                                    for jw in range(L2W // 16):
                                        scaled[par][
                                            eg, pl.ds(jw * 16, 16)] = (
                                            rows[par][eg,
                                                      pl.ds(jw * 16, 16)]
                                            * a)

                            pltpu.sync_copy(scaled[par],
                                            acc.at[dstv2.at[jj]],
                                            add=True)

            plsc.subcore_barrier()

            @pl.loop(0, NT // 40)
            def _(k5):
                pltpu.sync_copy(acc.at[pl.ds(s * NT + k5 * 40, 40)],
                                rows0.at[pl.ds(0, 40)])
                pltpu.sync_copy(rows0.at[pl.ds(0, 40)],
                                r_ref.at[pl.ds(s * NT + k5 * 40, 40)])

        @pl.when(c == 0)
        def _():
            work(0, ra_ref)

        @pl.when(c == 1)
        def _():
            work(1, rb_ref)

    return k(srcs, dsts, ee, f2p, z64)


def _sc_count(train_nodes, z1):
    @functools.partial(
        pl.kernel,
        out_type=jax.ShapeDtypeStruct((N,), jnp.float32),
        mesh=_mesh(), **_CP,
        scratch_types=[
            pltpu.VMEM((NTRAIN,), jnp.int32),
            pltpu.VMEM((5008,), jnp.float32),
            pltpu.VMEM((N,), jnp.float32),
            pltpu.VMEM_SHARED((N,), jnp.float32),
        ],
    )
    def k(tn_ref, z1_ref, cnt_ref, tnv, ones, zstage, cacc):
        c = lax.axis_index("c")
        s = lax.axis_index("s")

        @pl.when((c == 0) & (s == 0))
        def _():
            pltpu.sync_copy(z1_ref, zstage)
            pltpu.sync_copy(zstage, cacc)
            pltpu.sync_copy(tn_ref, tnv)

            @pl.loop(0, 313)
            def _(i):
                ones[pl.ds(i * 16, 16)] = jnp.ones((16,), jnp.float32)

            pltpu.sync_copy(ones.at[pl.ds(0, NTRAIN)], cacc.at[tnv],
                            add=True)
            pltpu.sync_copy(cacc, zstage)
            pltpu.sync_copy(zstage, cnt_ref)

    return k(train_nodes, z1)


def _blockdiag(a, hh, dh, fw):
    eye = jnp.eye(hh, dtype=a.dtype)
    out = (a[:, None, :] * eye[:, :, None]).reshape(hh, hh * dh)
    if out.shape[1] < fw:
        out = jnp.pad(out, ((0, 0), (0, fw - out.shape[1])))
    return out


def kernel(feats, edge_index, label, train_nodes, W0, al0, ar0, b0,
           W1, al1, ar1, b1, W2, al2, ar2, b2):
    edge = edge_index.astype(jnp.int32)
    srcs = edge[0]
    dsts = edge[1]

    alx0 = _blockdiag(al0, HEADS, D_HEAD, HID)
    arx0 = _blockdiag(ar0, HEADS, D_HEAD, HID)
    alx1 = _blockdiag(al1, HEADS, D_HEAD, HID)
    arx1 = _blockdiag(ar1, HEADS, D_HEAD, HID)
    W2x = jnp.pad(W2, ((0, 0), (0, L2W - OUTC)))
    alx2 = jnp.pad(al2, ((0, 0), (0, L2W - OUTC)))
    arx2 = jnp.pad(ar2, ((0, 0), (0, L2W - OUTC)))
    b2x = jnp.pad(b2, (0, L2W - OUTC)).reshape(1, L2W)
    b0r = b0.reshape(1, HID)
    b1r = b1.reshape(1, HID)

    featsp = jnp.pad(feats, ((0, N - NR), (0, 0)))
    labelp = jnp.pad(label.astype(jnp.int32), (0, N - NR))

    z4 = jnp.zeros((4 * N,), jnp.float32)
    z128 = jnp.zeros((N, 128), jnp.float32)
    z1 = jnp.zeros((N,), jnp.float32)
    z64 = jnp.zeros((N, L2W), jnp.float32)

    fa, fb, ela, elb, era, erb, m = _tc_entry(False, HEADS, HID, D_IN,
                                              (featsp,), W0, alx0, arx0)
    ee0, ee1, d0, d1 = _sc_logits8(ela.reshape(4 * N), elb.reshape(4 * N),
                                   era.reshape(4 * N), erb.reshape(4 * N),
                                   m, srcs, dsts, z4)
    ra, rb = _sc_aggr8(srcs, dsts, ee0, ee1, fa, fb, z128)

    fa, fb, ela, elb, era, erb, m = _tc_entry(True, HEADS, HID, D_IN,
                                              (ra, rb, d0.reshape(N, 4),
                                               d1.reshape(N, 4), b0r),
                                              W1, alx1, arx1)
    ee0, ee1, d0, d1 = _sc_logits8(ela.reshape(4 * N), elb.reshape(4 * N),
                                   era.reshape(4 * N), erb.reshape(4 * N),
                                   m, srcs, dsts, z4)
    ra, rb = _sc_aggr8(srcs, dsts, ee0, ee1, fa, fb, z128)

    f2, elt, ert, m = _tc_entry(True, 1, L2W, D_IN,
                                (ra, rb, d0.reshape(N, 4),
                                 d1.reshape(N, 4), b1r),
                                W2x, alx2, arx2)
    ee, dn0, dn1 = _sc_logits1(elt.reshape(N), ert.reshape(N), m,
                               srcs, dsts, z1)
    ra, rb = _sc_aggr1(srcs, dsts, ee, f2, z64)

    cnt = _sc_count(train_nodes.astype(jnp.int32), z1)

    logp, loss = _tc_final(ra, rb, dn0.reshape(N, 1), dn1.reshape(N, 1),
                           b2x, labelp.reshape(N, 1), cnt.reshape(N, 1))
    return logp[:NR], loss[0, 0]
